# Initial kernel scaffold; baseline (speedup 1.0000x reference)
import jax
import jax.numpy as jnp
from jax.experimental import pallas as pl


def kernel(x, edge_index, dist_rbf, eig_pe, edge_attr, batch, W_phi1, b_phi1, W_phi2, b_phi2, W_edge, W_rho, b_rho, Wq, Wk, Wv, We, Wr1, br1, Wr2, br2):
    raise NotImplementedError("write your pallas kernel here")


# speedup vs baseline: 12.2130x; 12.2130x over previous
import functools

import jax
import jax.numpy as jnp
from jax import lax
from jax.experimental import pallas as pl
from jax.experimental.pallas import tpu as pltpu
from jax.experimental.pallas import tpu_sc as plsc

F32 = jnp.float32
HI = jax.lax.Precision.HIGHEST

NC = 2
NS = 16
NW = NC * NS


def _dot(a, b):
    return jax.lax.dot(a, b, precision=HI, preferred_element_type=F32)


def _tc_signnet_h(pe, w1, b1, w2, b2):
    n = pe.shape[0]

    def body(pe_r, w1_r, b1_r, w2_r, b2_r, o_r):
        def phi(z):
            h1 = jnp.maximum(_dot(z, w1_r[...]) + b1_r[...], 0.0)
            return jnp.maximum(_dot(h1, w2_r[...]) + b2_r[...], 0.0)
        z = pe_r[...]
        o_r[...] = phi(z) + phi(-z)

    return pl.pallas_call(
        body,
        out_shape=jax.ShapeDtypeStruct((n, 128), F32),
    )(pe, w1, b1, w2, b2)


def _tc_msg(hs, edge_attr, w_edge, be=3200):
    e = hs.shape[0]

    def body(hs_r, ea_r, w_r, o_r):
        er = jnp.maximum(_dot(ea_r[...], w_r[...]), 0.0)
        o_r[...] = jnp.maximum(hs_r[...] + er, 0.0)

    return pl.pallas_call(
        body,
        grid=(e // be,),
        in_specs=[
            pl.BlockSpec((be, 128), lambda i: (i, 0)),
            pl.BlockSpec((be, 16), lambda i: (i, 0)),
            pl.BlockSpec((16, 128), lambda i: (0, 0)),
        ],
        out_specs=pl.BlockSpec((be, 128), lambda i: (i, 0)),
        out_shape=jax.ShapeDtypeStruct((e, 128), F32),
    )(hs, edge_attr, w_edge)


def _tc_qkv(x, h, m_part, w_rho, b_rho, wq, wk, wv, bn=2000):
    n = x.shape[0]

    def body(x_r, h_r, mp_r, wr_r, br_r, wq_r, wk_r, wv_r, q_r, k_r, v_r):
        hm = h_r[...] + mp_r[0] + mp_r[1]
        pos = _dot(hm, wr_r[...]) + br_r[...]
        hn = x_r[...] + pos
        q_r[...] = _dot(hn, wq_r[...])
        k_r[...] = _dot(hn, wk_r[...])
        v_r[...] = _dot(hn, wv_r[...])

    spec = pl.BlockSpec((bn, 128), lambda i: (i, 0))
    wspec = pl.BlockSpec((128, 128), lambda i: (0, 0))
    return pl.pallas_call(
        body,
        grid=(n // bn,),
        in_specs=[
            spec,
            spec,
            pl.BlockSpec((2, bn, 128), lambda i: (0, i, 0)),
            wspec,
            pl.BlockSpec((1, 128), lambda i: (0, 0)),
            wspec, wspec, wspec,
        ],
        out_specs=[spec, spec, spec],
        out_shape=[jax.ShapeDtypeStruct((n, 128), F32)] * 3,
    )(x, h, m_part, w_rho, b_rho, wq, wk, wv)


def _tc_score(qd, ks, dist_rbf, we, be=3200):
    e = qd.shape[0]

    def body(qd_r, ks_r, rbf_r, we_r, o_r):
        eb = _dot(rbf_r[...], we_r[...])
        p = qd_r[...] * (ks_r[...] + eb)
        j = lax.broadcasted_iota(jnp.int32, (128, 16), 0)
        hcol = lax.broadcasted_iota(jnp.int32, (128, 16), 1)
        t = jnp.where((j // 16) == hcol, 0.25, 0.0).astype(F32)
        o_r[...] = _dot(p, t)

    return pl.pallas_call(
        body,
        grid=(e // be,),
        in_specs=[
            pl.BlockSpec((be, 128), lambda i: (i, 0)),
            pl.BlockSpec((be, 128), lambda i: (i, 0)),
            pl.BlockSpec((be, 16), lambda i: (i, 0)),
            pl.BlockSpec((16, 128), lambda i: (0, 0)),
        ],
        out_specs=pl.BlockSpec((be, 16), lambda i: (i, 0)),
        out_shape=jax.ShapeDtypeStruct((e, 16), F32),
    )(qd, ks, dist_rbf, we)


def _tc_smax_merge(smax_part, bm=625):
    npk = smax_part.shape[1]

    def body(sp_r, o_r):
        pk = jnp.max(sp_r[...], axis=0)
        z = jnp.zeros((pk.shape[0], 120), F32)
        parts = [
            jnp.concatenate([pk[:, g * 8:(g + 1) * 8], z], axis=1)[:, None, :]
            for g in range(16)
        ]
        o_r[...] = jnp.concatenate(parts, axis=1)

    return pl.pallas_call(
        body,
        grid=(npk // bm,),
        in_specs=[pl.BlockSpec((NW, bm, 128), lambda i: (0, i, 0))],
        out_specs=pl.BlockSpec((bm, 16, 128), lambda i: (i, 0, 0)),
        out_shape=jax.ShapeDtypeStruct((npk, 16, 128), F32),
    )(smax_part)


def _tc_softmax_w(score, sm_e, vs, be=3200):
    e = score.shape[0]

    def body(s_r, sm_r, vs_r, wv_r, ex_r):
        ex = jnp.exp(s_r[...] - sm_r[:, :16])
        hrow = lax.broadcasted_iota(jnp.int32, (16, 128), 0)
        j = lax.broadcasted_iota(jnp.int32, (16, 128), 1)
        r = jnp.where((j // 16) == hrow, 1.0, 0.0).astype(F32)
        exb = _dot(ex, r)
        wv_r[...] = vs_r[...] * exb
        ex_r[...] = jnp.concatenate(
            [ex, jnp.zeros((ex.shape[0], 112), F32)], axis=1)

    return pl.pallas_call(
        body,
        grid=(e // be,),
        in_specs=[
            pl.BlockSpec((be, 16), lambda i: (i, 0)),
            pl.BlockSpec((be, 128), lambda i: (i, 0)),
            pl.BlockSpec((be, 128), lambda i: (i, 0)),
        ],
        out_specs=[
            pl.BlockSpec((be, 128), lambda i: (i, 0)),
            pl.BlockSpec((be, 128), lambda i: (i, 0)),
        ],
        out_shape=[
            jax.ShapeDtypeStruct((e, 128), F32),
            jax.ShapeDtypeStruct((e, 128), F32),
        ],
    )(score, sm_e, vs)


def _tc_final(agg_part, den_part, batch3, wr1, br1, wr2, br2, bn=1000):
    n = agg_part.shape[1]
    nblk = n // bn

    def body(ap_r, dp_r, b_r, wr1_r, br1_r, wr2_r, br2_r, o_r, gsum, cnt):
        i = pl.program_id(0)

        @pl.when(i == 0)
        def _():
            gsum[...] = jnp.zeros_like(gsum)
            cnt[...] = jnp.zeros_like(cnt)

        a = ap_r[0] + ap_r[1]
        dsum = dp_r[0, :, :16] + dp_r[1, :, :16]
        hrow = lax.broadcasted_iota(jnp.int32, (16, 128), 0)
        j = lax.broadcasted_iota(jnp.int32, (16, 128), 1)
        r = jnp.where((j // 16) == hrow, 1.0, 0.0).astype(F32)
        den = _dot(dsum, r) + 1e-9
        node = a / den

        ids = b_r[0]
        g = lax.broadcasted_iota(jnp.int32, (128, bn), 0)
        oh = (jnp.broadcast_to(ids, (128, bn)) == g).astype(F32)
        gsum[...] += _dot(oh, node)
        cnt[...] += jnp.sum(oh, axis=1, keepdims=True)

        @pl.when(i == nblk - 1)
        def _():
            graph = gsum[...] / jnp.maximum(cnt[...], 1.0)
            og = jnp.maximum(_dot(graph, wr1_r[...]) + br1_r[...], 0.0)
            o_r[...] = _dot(og, wr2_r[...]) + br2_r[...]

    return pl.pallas_call(
        body,
        grid=(nblk,),
        in_specs=[
            pl.BlockSpec((2, bn, 128), lambda i: (0, i, 0)),
            pl.BlockSpec((2, bn, 128), lambda i: (0, i, 0)),
            pl.BlockSpec((1, 1, bn), lambda i: (i, 0, 0)),
            pl.BlockSpec((128, 128), lambda i: (0, 0)),
            pl.BlockSpec((1, 128), lambda i: (0, 0)),
            pl.BlockSpec((128, 1), lambda i: (0, 0)),
            pl.BlockSpec((1, 1), lambda i: (0, 0)),
        ],
        out_specs=pl.BlockSpec((128, 1), lambda i: (0, 0)),
        out_shape=jax.ShapeDtypeStruct((128, 1), F32),
        scratch_shapes=[
            pltpu.VMEM((128, 128), F32),
            pltpu.VMEM((128, 1), F32),
        ],
    )(agg_part, den_part, batch3, wr1, br1, wr2, br2)


_MESH = plsc.VectorSubcoreMesh(core_axis_name="c", subcore_axis_name="s")


def _sc_gather(tables, idxs, chunk):
    e = idxs[0].shape[0]
    per_w = e // NW
    n_chunks = per_w // chunk
    t_count = len(tables)
    widths = [t.shape[1] for t in tables]

    scratch = []
    for w in widths:
        scratch.append(pltpu.VMEM((chunk,), jnp.int32))
        scratch.append(pltpu.VMEM((chunk, w), F32))

    @functools.partial(
        pl.kernel,
        out_type=[jax.ShapeDtypeStruct((e, w), F32) for w in widths],
        mesh=_MESH,
        scratch_types=scratch,
    )
    def k(*refs):
        tabs = refs[:t_count]
        idx_hbm = refs[t_count:2 * t_count]
        outs = refs[2 * t_count:3 * t_count]
        bufs = refs[3 * t_count:]
        cid = lax.axis_index("c")
        sid = lax.axis_index("s")
        base = (cid * NS + sid) * per_w

        @pl.loop(0, n_chunks)
        def _(ci):
            off = base + ci * chunk
            for t in range(t_count):
                ib = bufs[2 * t]
                rb = bufs[2 * t + 1]
                pltpu.sync_copy(idx_hbm[t].at[pl.ds(off, chunk)], ib)
                pltpu.sync_copy(tabs[t].at[ib], rb)
                pltpu.sync_copy(rb, outs[t].at[pl.ds(off, chunk)])

    res = k(*tables, *idxs)
    if not isinstance(res, (list, tuple)):
        res = [res]
    return list(res)


def _sc_scatter_add(values, idx, n, chunk=80):
    e, width = values.shape
    e2 = e // NC
    per_w = e2 // NS
    n_chunks = per_w // chunk
    npad = 10240
    rows_per_sub = npad // NS
    zrows = rows_per_sub // 5

    @functools.partial(
        pl.kernel,
        out_type=jax.ShapeDtypeStruct((NC, npad, width), F32),
        mesh=_MESH,
        scratch_types=[
            pltpu.VMEM((chunk,), jnp.int32),
            pltpu.VMEM((chunk, width), F32),
            pltpu.VMEM((zrows, width), F32),
            pltpu.VMEM_SHARED((npad, width), F32),
        ],
    )
    def k(val_hbm, idx_hbm, out_hbm, idxbuf, valbuf, zbuf, acc):
        cid = lax.axis_index("c")
        sid = lax.axis_index("s")

        @pl.loop(0, zrows)
        def _(r):
            for cc in range(width // 16):
                zbuf[r, pl.ds(cc * 16, 16)] = jnp.zeros((16,), F32)

        for kk in range(5):
            pltpu.sync_copy(
                zbuf, acc.at[pl.ds(sid * rows_per_sub + kk * zrows, zrows)])
        plsc.subcore_barrier()

        base = cid * e2 + sid * per_w

        @pl.loop(0, n_chunks)
        def _(ci):
            off = base + ci * chunk
            pltpu.sync_copy(idx_hbm.at[pl.ds(off, chunk)], idxbuf)
            pltpu.sync_copy(val_hbm.at[pl.ds(off, chunk)], valbuf)
            pltpu.sync_copy(valbuf, acc.at[idxbuf], add=True)

        plsc.subcore_barrier()
        for kk in range(5):
            rs = sid * rows_per_sub + kk * zrows
            pltpu.sync_copy(acc.at[pl.ds(rs, zrows)],
                            out_hbm.at[cid].at[pl.ds(rs, zrows)])

    return k(values, idx)[:, :n, :]


def _sc_segmax(score_flat, idx, n, chunk=1000):
    e = idx.shape[0]
    per_w = e // NW
    n_chunks = per_w // chunk
    tsz = n * 8 + 16

    @functools.partial(
        pl.kernel,
        out_type=jax.ShapeDtypeStruct((NW * n * 8,), F32),
        mesh=_MESH,
        scratch_types=[
            pltpu.VMEM((chunk + 16,), jnp.int32),
            pltpu.VMEM((chunk * 16,), F32),
            pltpu.VMEM((tsz,), F32),
        ],
        compiler_params=pltpu.CompilerParams(needs_layout_passes=False),
    )
    def k(score_hbm, idx_hbm, out_hbm, idxbuf, scorebuf, table):
        cid = lax.axis_index("c")
        sid = lax.axis_index("s")
        wid = cid * NS + sid
        base = wid * per_w

        @pl.loop(0, tsz, step=16)
        def _(i):
            table[pl.ds(i, 16)] = jnp.full((16,), -3.0e38, F32)

        lanes = lax.iota(jnp.int32, 16)
        mask = lanes < 8

        @pl.loop(0, n_chunks)
        def _(ci):
            off = base + ci * chunk
            pltpu.sync_copy(score_hbm.at[pl.ds(off * 16, chunk * 16)],
                            scorebuf)
            pltpu.sync_copy(idx_hbm.at[pl.ds(off, chunk)],
                            idxbuf.at[pl.ds(0, chunk)])

            @pl.loop(0, chunk)
            def _(ei):
                d = idxbuf[pl.ds(ei, 16)][0]
                sv = scorebuf[pl.ds(ei * 16, 16)]
                iv = d * 8 + lanes
                cur = plsc.load_gather(table, [iv], mask=mask)
                plsc.store_scatter(table, [iv],
                                   jnp.maximum(cur, sv), mask=mask)

        pltpu.sync_copy(table.at[pl.ds(0, n * 8)],
                        out_hbm.at[pl.ds(wid * n * 8, n * 8)])

    return k(score_flat, idx)


def kernel(x, edge_index, dist_rbf, eig_pe, edge_attr, batch,
           W_phi1, b_phi1, W_phi2, b_phi2, W_edge, W_rho, b_rho,
           Wq, Wk, Wv, We, Wr1, br1, Wr2, br2):
    n = x.shape[0]
    e = edge_index.shape[1]
    src = edge_index[0]
    dst = edge_index[1]
    b_phi1 = b_phi1.reshape(1, -1)
    b_phi2 = b_phi2.reshape(1, -1)
    b_rho = b_rho.reshape(1, -1)
    br1 = br1.reshape(1, -1)
    br2 = br2.reshape(1, 1)
    batch3 = batch.reshape(n // 1000, 1, 1000)

    h = _tc_signnet_h(eig_pe, W_phi1, b_phi1, W_phi2, b_phi2)
    (hs,) = _sc_gather([h], [src], chunk=80)
    msg = _tc_msg(hs, edge_attr, W_edge)
    m_part = _sc_scatter_add(msg, dst, n)
    q, k, v = _tc_qkv(x, h, m_part, W_rho, b_rho, Wq, Wk, Wv)
    qd, ks, vs = _sc_gather([q, k, v], [dst, src, src], chunk=80)
    score = _tc_score(qd, ks, dist_rbf, We)
    smax_part = _sc_segmax(score.reshape(-1), dst, n)
    smax_pad = _tc_smax_merge(
        smax_part.reshape(NW, n // 16, 128)).reshape(n, 128)
    (sm_e,) = _sc_gather([smax_pad], [dst], chunk=80)
    wv, ex = _tc_softmax_w(score, sm_e, vs)
    agg_part = _sc_scatter_add(wv, dst, n)
    den_part = _sc_scatter_add(ex, dst, n)
    out = _tc_final(agg_part, den_part, batch3, Wr1, br1, Wr2, br2)
    return out


# speedup vs baseline: 12.4351x; 1.0182x over previous
import functools

import jax
import jax.numpy as jnp
from jax import lax
from jax.experimental import pallas as pl
from jax.experimental.pallas import tpu as pltpu
from jax.experimental.pallas import tpu_sc as plsc

F32 = jnp.float32
HI = jax.lax.Precision.HIGHEST

NC = 2
NS = 16
NW = NC * NS


def _dot(a, b):
    return jax.lax.dot(a, b, precision=HI, preferred_element_type=F32)


def _tc_signnet_h(pe, w1, b1, w2, b2):
    n = pe.shape[0]

    def body(pe_r, w1_r, b1_r, w2_r, b2_r, o_r):
        def phi(z):
            h1 = jnp.maximum(_dot(z, w1_r[...]) + b1_r[...], 0.0)
            return jnp.maximum(_dot(h1, w2_r[...]) + b2_r[...], 0.0)
        z = pe_r[...]
        o_r[...] = phi(z) + phi(-z)

    return pl.pallas_call(
        body,
        out_shape=jax.ShapeDtypeStruct((n, 128), F32),
    )(pe, w1, b1, w2, b2)


def _tc_msg(hs, edge_attr, w_edge, be=3200):
    e = hs.shape[0]

    def body(hs_r, ea_r, w_r, o_r):
        er = jnp.maximum(_dot(ea_r[...], w_r[...]), 0.0)
        o_r[...] = jnp.maximum(hs_r[...] + er, 0.0)

    return pl.pallas_call(
        body,
        grid=(e // be,),
        in_specs=[
            pl.BlockSpec((be, 128), lambda i: (i, 0)),
            pl.BlockSpec((be, 16), lambda i: (i, 0)),
            pl.BlockSpec((16, 128), lambda i: (0, 0)),
        ],
        out_specs=pl.BlockSpec((be, 128), lambda i: (i, 0)),
        out_shape=jax.ShapeDtypeStruct((e, 128), F32),
    )(hs, edge_attr, w_edge)


def _tc_qkv(x, h, m_part, w_rho, b_rho, wq, wk, wv, bn=2000):
    n = x.shape[0]

    def body(x_r, h_r, mp_r, wr_r, br_r, wq_r, wk_r, wv_r, q_r, k_r, v_r):
        hm = h_r[...] + mp_r[0] + mp_r[1]
        pos = _dot(hm, wr_r[...]) + br_r[...]
        hn = x_r[...] + pos
        q_r[...] = _dot(hn, wq_r[...])
        k_r[...] = _dot(hn, wk_r[...])
        v_r[...] = _dot(hn, wv_r[...])

    spec = pl.BlockSpec((bn, 128), lambda i: (i, 0))
    wspec = pl.BlockSpec((128, 128), lambda i: (0, 0))
    return pl.pallas_call(
        body,
        grid=(n // bn,),
        in_specs=[
            spec,
            spec,
            pl.BlockSpec((2, bn, 128), lambda i: (0, i, 0)),
            wspec,
            pl.BlockSpec((1, 128), lambda i: (0, 0)),
            wspec, wspec, wspec,
        ],
        out_specs=[spec, spec, spec],
        out_shape=[jax.ShapeDtypeStruct((n, 128), F32)] * 3,
    )(x, h, m_part, w_rho, b_rho, wq, wk, wv)


def _tc_score(qd, ks, dist_rbf, we, be=3200):
    e = qd.shape[0]

    def body(qd_r, ks_r, rbf_r, we_r, o_r):
        eb = _dot(rbf_r[...], we_r[...])
        p = qd_r[...] * (ks_r[...] + eb)
        j = lax.broadcasted_iota(jnp.int32, (128, 16), 0)
        hcol = lax.broadcasted_iota(jnp.int32, (128, 16), 1)
        t = jnp.where((j // 16) == hcol, 0.25, 0.0).astype(F32)
        o_r[...] = _dot(p, t)

    return pl.pallas_call(
        body,
        grid=(e // be,),
        in_specs=[
            pl.BlockSpec((be, 128), lambda i: (i, 0)),
            pl.BlockSpec((be, 128), lambda i: (i, 0)),
            pl.BlockSpec((be, 16), lambda i: (i, 0)),
            pl.BlockSpec((16, 128), lambda i: (0, 0)),
        ],
        out_specs=pl.BlockSpec((be, 16), lambda i: (i, 0)),
        out_shape=jax.ShapeDtypeStruct((e, 16), F32),
    )(qd, ks, dist_rbf, we)


def _tc_smax_merge(smax_part, bm=625):
    npk = smax_part.shape[1]

    def body(sp_r, o_r):
        pk = jnp.max(sp_r[...], axis=0)
        z = jnp.zeros((pk.shape[0], 120), F32)
        parts = [
            jnp.concatenate([pk[:, g * 8:(g + 1) * 8], z], axis=1)[:, None, :]
            for g in range(16)
        ]
        o_r[...] = jnp.concatenate(parts, axis=1)

    return pl.pallas_call(
        body,
        grid=(npk // bm,),
        in_specs=[pl.BlockSpec((NW, bm, 128), lambda i: (0, i, 0))],
        out_specs=pl.BlockSpec((bm, 16, 128), lambda i: (i, 0, 0)),
        out_shape=jax.ShapeDtypeStruct((npk, 16, 128), F32),
    )(smax_part)


def _tc_final(agg_part, den_part, batch3, wr1, br1, wr2, br2, bn=1000):
    n = agg_part.shape[1]
    nblk = n // bn

    def body(ap_r, dp_r, b_r, wr1_r, br1_r, wr2_r, br2_r, o_r, gsum, cnt):
        i = pl.program_id(0)

        @pl.when(i == 0)
        def _():
            gsum[...] = jnp.zeros_like(gsum)
            cnt[...] = jnp.zeros_like(cnt)

        a = ap_r[0] + ap_r[1]
        dsum = dp_r[0, :, :16] + dp_r[1, :, :16]
        hrow = lax.broadcasted_iota(jnp.int32, (16, 128), 0)
        j = lax.broadcasted_iota(jnp.int32, (16, 128), 1)
        r = jnp.where((j // 16) == hrow, 1.0, 0.0).astype(F32)
        den = _dot(dsum, r) + 1e-9
        node = a / den

        ids = b_r[0]
        g = lax.broadcasted_iota(jnp.int32, (128, bn), 0)
        oh = (jnp.broadcast_to(ids, (128, bn)) == g).astype(F32)
        gsum[...] += _dot(oh, node)
        cnt[...] += jnp.sum(oh, axis=1, keepdims=True)

        @pl.when(i == nblk - 1)
        def _():
            graph = gsum[...] / jnp.maximum(cnt[...], 1.0)
            og = jnp.maximum(_dot(graph, wr1_r[...]) + br1_r[...], 0.0)
            o_r[...] = _dot(og, wr2_r[...]) + br2_r[...]

    return pl.pallas_call(
        body,
        grid=(nblk,),
        in_specs=[
            pl.BlockSpec((2, bn, 128), lambda i: (0, i, 0)),
            pl.BlockSpec((2, bn, 128), lambda i: (0, i, 0)),
            pl.BlockSpec((1, 1, bn), lambda i: (i, 0, 0)),
            pl.BlockSpec((128, 128), lambda i: (0, 0)),
            pl.BlockSpec((1, 128), lambda i: (0, 0)),
            pl.BlockSpec((128, 1), lambda i: (0, 0)),
            pl.BlockSpec((1, 1), lambda i: (0, 0)),
        ],
        out_specs=pl.BlockSpec((128, 1), lambda i: (0, 0)),
        out_shape=jax.ShapeDtypeStruct((128, 1), F32),
        scratch_shapes=[
            pltpu.VMEM((128, 128), F32),
            pltpu.VMEM((128, 1), F32),
        ],
    )(agg_part, den_part, batch3, wr1, br1, wr2, br2)


_MESH = plsc.VectorSubcoreMesh(core_axis_name="c", subcore_axis_name="s")


def _sc_gather(tables, idxs, chunk):
    e = idxs[0].shape[0]
    per_w = e // NW
    n_chunks = per_w // chunk
    t_count = len(tables)
    widths = [t.shape[1] for t in tables]

    scratch = []
    for w in widths:
        scratch.append(pltpu.VMEM((chunk,), jnp.int32))
        scratch.append(pltpu.VMEM((chunk, w), F32))

    @functools.partial(
        pl.kernel,
        out_type=[jax.ShapeDtypeStruct((e, w), F32) for w in widths],
        mesh=_MESH,
        scratch_types=scratch,
    )
    def k(*refs):
        tabs = refs[:t_count]
        idx_hbm = refs[t_count:2 * t_count]
        outs = refs[2 * t_count:3 * t_count]
        bufs = refs[3 * t_count:]
        cid = lax.axis_index("c")
        sid = lax.axis_index("s")
        base = (cid * NS + sid) * per_w

        @pl.loop(0, n_chunks)
        def _(ci):
            off = base + ci * chunk
            for t in range(t_count):
                ib = bufs[2 * t]
                rb = bufs[2 * t + 1]
                pltpu.sync_copy(idx_hbm[t].at[pl.ds(off, chunk)], ib)
                pltpu.sync_copy(tabs[t].at[ib], rb)
                pltpu.sync_copy(rb, outs[t].at[pl.ds(off, chunk)])

    res = k(*tables, *idxs)
    if not isinstance(res, (list, tuple)):
        res = [res]
    return list(res)


def _sc_scatter_add(values, idx, n, chunk=80):
    e, width = values.shape
    e2 = e // NC
    per_w = e2 // NS
    n_chunks = per_w // chunk
    npad = 10240
    rows_per_sub = npad // NS
    zrows = rows_per_sub // 5

    @functools.partial(
        pl.kernel,
        out_type=jax.ShapeDtypeStruct((NC, npad, width), F32),
        mesh=_MESH,
        scratch_types=[
            pltpu.VMEM((chunk,), jnp.int32),
            pltpu.VMEM((chunk, width), F32),
            pltpu.VMEM((zrows, width), F32),
            pltpu.VMEM_SHARED((npad, width), F32),
        ],
    )
    def k(val_hbm, idx_hbm, out_hbm, idxbuf, valbuf, zbuf, acc):
        cid = lax.axis_index("c")
        sid = lax.axis_index("s")

        @pl.loop(0, zrows)
        def _(r):
            for cc in range(width // 16):
                zbuf[r, pl.ds(cc * 16, 16)] = jnp.zeros((16,), F32)

        for kk in range(5):
            pltpu.sync_copy(
                zbuf, acc.at[pl.ds(sid * rows_per_sub + kk * zrows, zrows)])
        plsc.subcore_barrier()

        base = cid * e2 + sid * per_w

        @pl.loop(0, n_chunks)
        def _(ci):
            off = base + ci * chunk
            pltpu.sync_copy(idx_hbm.at[pl.ds(off, chunk)], idxbuf)
            pltpu.sync_copy(val_hbm.at[pl.ds(off, chunk)], valbuf)
            pltpu.sync_copy(valbuf, acc.at[idxbuf], add=True)

        plsc.subcore_barrier()
        for kk in range(5):
            rs = sid * rows_per_sub + kk * zrows
            pltpu.sync_copy(acc.at[pl.ds(rs, zrows)],
                            out_hbm.at[cid].at[pl.ds(rs, zrows)])

    return k(values, idx)[:, :n, :]


def _sc_segmax(score_flat, idx, n, chunk=1000):
    e = idx.shape[0]
    per_w = e // NW
    n_chunks = per_w // chunk
    tsz = n * 8 + 16

    @functools.partial(
        pl.kernel,
        out_type=jax.ShapeDtypeStruct((NW * n * 8,), F32),
        mesh=_MESH,
        scratch_types=[
            pltpu.VMEM((chunk + 16,), jnp.int32),
            pltpu.VMEM((chunk * 16,), F32),
            pltpu.VMEM((tsz,), F32),
        ],
        compiler_params=pltpu.CompilerParams(needs_layout_passes=False),
    )
    def k(score_hbm, idx_hbm, out_hbm, idxbuf, scorebuf, table):
        cid = lax.axis_index("c")
        sid = lax.axis_index("s")
        wid = cid * NS + sid
        base = wid * per_w

        @pl.loop(0, tsz, step=16)
        def _(i):
            table[pl.ds(i, 16)] = jnp.full((16,), -3.0e38, F32)

        lanes = lax.iota(jnp.int32, 16)
        mask = lanes < 8

        @pl.loop(0, n_chunks)
        def _(ci):
            off = base + ci * chunk
            pltpu.sync_copy(score_hbm.at[pl.ds(off * 16, chunk * 16)],
                            scorebuf)
            pltpu.sync_copy(idx_hbm.at[pl.ds(off, chunk)],
                            idxbuf.at[pl.ds(0, chunk)])

            @pl.loop(0, chunk)
            def _(ei):
                d = idxbuf[pl.ds(ei, 16)][0]
                sv = scorebuf[pl.ds(ei * 16, 16)]
                iv = d * 8 + lanes
                cur = plsc.load_gather(table, [iv], mask=mask)
                plsc.store_scatter(table, [iv],
                                   jnp.maximum(cur, sv), mask=mask)

        pltpu.sync_copy(table.at[pl.ds(0, n * 8)],
                        out_hbm.at[pl.ds(wid * n * 8, n * 8)])

    return k(score_flat, idx)


def _sc_den_exp(score_flat, smax_pad, idx, n, chunk=80):
    e = idx.shape[0]
    e2 = e // NC
    per_w = e2 // NS
    n_chunks = per_w // chunk
    npad = 10240
    rows_per_sub = npad // NS
    zrows = rows_per_sub // 5

    @functools.partial(
        pl.kernel,
        out_type=[
            jax.ShapeDtypeStruct((NC, npad, 128), F32),
            jax.ShapeDtypeStruct((e * 16,), F32),
        ],
        mesh=_MESH,
        scratch_types=[
            pltpu.VMEM((chunk,), jnp.int32),
            pltpu.VMEM((chunk * 16,), F32),
            pltpu.VMEM((chunk, 128), F32),
            pltpu.VMEM((chunk, 128), F32),
            pltpu.VMEM((chunk * 16,), F32),
            pltpu.VMEM((zrows, 128), F32),
            pltpu.VMEM_SHARED((npad, 128), F32),
        ],
    )
    def k(score_hbm, smax_hbm, idx_hbm, out_hbm, ex_hbm,
          idxbuf, sbuf, smbuf, vbuf, exbuf, zbuf, acc):
        cid = lax.axis_index("c")
        sid = lax.axis_index("s")

        @pl.loop(0, zrows)
        def _(r):
            for cc in range(8):
                zbuf[r, pl.ds(cc * 16, 16)] = jnp.zeros((16,), F32)

        for kk in range(5):
            pltpu.sync_copy(
                zbuf, acc.at[pl.ds(sid * rows_per_sub + kk * zrows, zrows)])
        plsc.subcore_barrier()

        @pl.loop(0, chunk)
        def _(r):
            for cc in range(1, 8):
                vbuf[r, pl.ds(cc * 16, 16)] = jnp.zeros((16,), F32)

        base = cid * e2 + sid * per_w

        @pl.loop(0, n_chunks)
        def _(ci):
            off = base + ci * chunk
            pltpu.sync_copy(idx_hbm.at[pl.ds(off, chunk)], idxbuf)
            pltpu.sync_copy(score_hbm.at[pl.ds(off * 16, chunk * 16)], sbuf)
            pltpu.sync_copy(smax_hbm.at[idxbuf], smbuf)

            @pl.loop(0, chunk)
            def _(ei):
                s = sbuf[pl.ds(ei * 16, 16)]
                m = smbuf[ei, pl.ds(0, 16)]
                ex = jnp.exp(s - m)
                vbuf[ei, pl.ds(0, 16)] = ex
                exbuf[pl.ds(ei * 16, 16)] = ex

            pltpu.sync_copy(vbuf, acc.at[idxbuf], add=True)
            pltpu.sync_copy(exbuf, ex_hbm.at[pl.ds(off * 16, chunk * 16)])

        plsc.subcore_barrier()
        for kk in range(5):
            rs = sid * rows_per_sub + kk * zrows
            pltpu.sync_copy(acc.at[pl.ds(rs, zrows)],
                            out_hbm.at[cid].at[pl.ds(rs, zrows)])

    den, ex_flat = k(score_flat, smax_pad, idx)
    return den[:, :n, :], ex_flat


def _tc_wv(ex_pk, vs3, be=3200):
    ep8 = ex_pk.shape[0]
    bp = be // 8

    def body(p_r, vs_r, o_r):
        p = p_r[...]
        a = lax.broadcasted_iota(jnp.int32, (128, 128), 0)
        j = lax.broadcasted_iota(jnp.int32, (128, 128), 1)
        parts = []
        for g in range(8):
            m = (a == g * 16 + j // 16).astype(F32)
            parts.append(_dot(p, m)[:, None, :])
        exb = jnp.concatenate(parts, axis=1)
        o_r[...] = vs_r[...] * exb

    return pl.pallas_call(
        body,
        grid=(ep8 // bp,),
        in_specs=[
            pl.BlockSpec((bp, 128), lambda i: (i, 0)),
            pl.BlockSpec((bp, 8, 128), lambda i: (i, 0, 0)),
        ],
        out_specs=pl.BlockSpec((bp, 8, 128), lambda i: (i, 0, 0)),
        out_shape=jax.ShapeDtypeStruct((ep8, 8, 128), F32),
    )(ex_pk, vs3)


def kernel(x, edge_index, dist_rbf, eig_pe, edge_attr, batch,
           W_phi1, b_phi1, W_phi2, b_phi2, W_edge, W_rho, b_rho,
           Wq, Wk, Wv, We, Wr1, br1, Wr2, br2):
    n = x.shape[0]
    e = edge_index.shape[1]
    src = edge_index[0]
    dst = edge_index[1]
    b_phi1 = b_phi1.reshape(1, -1)
    b_phi2 = b_phi2.reshape(1, -1)
    b_rho = b_rho.reshape(1, -1)
    br1 = br1.reshape(1, -1)
    br2 = br2.reshape(1, 1)
    batch3 = batch.reshape(n // 1000, 1, 1000)

    h = _tc_signnet_h(eig_pe, W_phi1, b_phi1, W_phi2, b_phi2)
    (hs,) = _sc_gather([h], [src], chunk=80)
    msg = _tc_msg(hs, edge_attr, W_edge)
    m_part = _sc_scatter_add(msg, dst, n)
    q, k, v = _tc_qkv(x, h, m_part, W_rho, b_rho, Wq, Wk, Wv)
    qd, ks, vs = _sc_gather([q, k, v], [dst, src, src], chunk=80)
    score = _tc_score(qd, ks, dist_rbf, We)
    score_flat = score.reshape(-1)
    smax_part = _sc_segmax(score_flat, dst, n)
    smax_pad = _tc_smax_merge(
        smax_part.reshape(NW, n // 16, 128)).reshape(n, 128)
    den_part, ex_flat = _sc_den_exp(score_flat, smax_pad, dst, n)
    wv = _tc_wv(ex_flat.reshape(e // 8, 128),
                vs.reshape(e // 8, 8, 128)).reshape(e, 128)
    agg_part = _sc_scatter_add(wv, dst, n)
    out = _tc_final(agg_part, den_part, batch3, Wr1, br1, Wr2, br2)
    return out


# speedup vs baseline: 13.7999x; 1.1098x over previous
import functools

import jax
import jax.numpy as jnp
from jax import lax
from jax.experimental import pallas as pl
from jax.experimental.pallas import tpu as pltpu
from jax.experimental.pallas import tpu_sc as plsc

F32 = jnp.float32
HI = jax.lax.Precision.HIGHEST

NC = 2
NS = 16
NW = NC * NS


def _dot(a, b):
    return jax.lax.dot(a, b, precision=HI, preferred_element_type=F32)


def _tc_signnet_h(pe, w1, b1, w2, b2):
    n = pe.shape[0]

    def body(pe_r, w1_r, b1_r, w2_r, b2_r, o_r):
        def phi(z):
            h1 = jnp.maximum(_dot(z, w1_r[...]) + b1_r[...], 0.0)
            return jnp.maximum(_dot(h1, w2_r[...]) + b2_r[...], 0.0)
        z = pe_r[...]
        o_r[...] = phi(z) + phi(-z)

    return pl.pallas_call(
        body,
        out_shape=jax.ShapeDtypeStruct((n, 128), F32),
    )(pe, w1, b1, w2, b2)


def _tc_msg(hs, edge_attr, w_edge, be=3200):
    e = hs.shape[0]

    def body(hs_r, ea_r, w_r, o_r):
        er = jnp.maximum(_dot(ea_r[...], w_r[...]), 0.0)
        o_r[...] = jnp.maximum(hs_r[...] + er, 0.0)

    return pl.pallas_call(
        body,
        grid=(e // be,),
        in_specs=[
            pl.BlockSpec((be, 128), lambda i: (i, 0)),
            pl.BlockSpec((be, 16), lambda i: (i, 0)),
            pl.BlockSpec((16, 128), lambda i: (0, 0)),
        ],
        out_specs=pl.BlockSpec((be, 128), lambda i: (i, 0)),
        out_shape=jax.ShapeDtypeStruct((e, 128), F32),
    )(hs, edge_attr, w_edge)


def _tc_qkv(x, h, m_part, w_rho, b_rho, wq, wk, wv, bn=2000):
    n = x.shape[0]

    def body(x_r, h_r, mp_r, wr_r, br_r, wq_r, wk_r, wv_r, q_r, k_r, v_r):
        hm = h_r[...] + mp_r[0] + mp_r[1]
        pos = _dot(hm, wr_r[...]) + br_r[...]
        hn = x_r[...] + pos
        q_r[...] = _dot(hn, wq_r[...])
        k_r[...] = _dot(hn, wk_r[...])
        v_r[...] = _dot(hn, wv_r[...])

    spec = pl.BlockSpec((bn, 128), lambda i: (i, 0))
    wspec = pl.BlockSpec((128, 128), lambda i: (0, 0))
    return pl.pallas_call(
        body,
        grid=(n // bn,),
        in_specs=[
            spec,
            spec,
            pl.BlockSpec((2, bn, 128), lambda i: (0, i, 0)),
            wspec,
            pl.BlockSpec((1, 128), lambda i: (0, 0)),
            wspec, wspec, wspec,
        ],
        out_specs=[spec, spec, spec],
        out_shape=[jax.ShapeDtypeStruct((n, 128), F32)] * 3,
    )(x, h, m_part, w_rho, b_rho, wq, wk, wv)


def _tc_score(qd, ks, dist_rbf, we, be=3200):
    e = qd.shape[0]

    def body(qd_r, ks_r, rbf_r, we_r, o_r):
        eb = _dot(rbf_r[...], we_r[...])
        p = qd_r[...] * (ks_r[...] + eb)
        j = lax.broadcasted_iota(jnp.int32, (128, 16), 0)
        hcol = lax.broadcasted_iota(jnp.int32, (128, 16), 1)
        t = jnp.where((j // 16) == hcol, 0.25, 0.0).astype(F32)
        o_r[...] = _dot(p, t)

    return pl.pallas_call(
        body,
        grid=(e // be,),
        in_specs=[
            pl.BlockSpec((be, 128), lambda i: (i, 0)),
            pl.BlockSpec((be, 128), lambda i: (i, 0)),
            pl.BlockSpec((be, 16), lambda i: (i, 0)),
            pl.BlockSpec((16, 128), lambda i: (0, 0)),
        ],
        out_specs=pl.BlockSpec((be, 16), lambda i: (i, 0)),
        out_shape=jax.ShapeDtypeStruct((e, 16), F32),
    )(qd, ks, dist_rbf, we)


def _tc_smax_merge(smax_part, bm=625):
    npk = smax_part.shape[1]

    def body(sp_r, o_r):
        pk = jnp.max(sp_r[...], axis=0)
        z = jnp.zeros((pk.shape[0], 120), F32)
        parts = [
            jnp.concatenate([pk[:, g * 8:(g + 1) * 8], z], axis=1)[:, None, :]
            for g in range(16)
        ]
        o_r[...] = jnp.concatenate(parts, axis=1)

    return pl.pallas_call(
        body,
        grid=(npk // bm,),
        in_specs=[pl.BlockSpec((NW, bm, 128), lambda i: (0, i, 0))],
        out_specs=pl.BlockSpec((bm, 16, 128), lambda i: (i, 0, 0)),
        out_shape=jax.ShapeDtypeStruct((npk, 16, 128), F32),
    )(smax_part)


def _tc_final(agg_part, den_part, batch3, wr1, br1, wr2, br2, bn=1000):
    n = agg_part.shape[1]
    nblk = n // bn

    def body(ap_r, dp_r, b_r, wr1_r, br1_r, wr2_r, br2_r, o_r, gsum, cnt):
        i = pl.program_id(0)

        @pl.when(i == 0)
        def _():
            gsum[...] = jnp.zeros_like(gsum)
            cnt[...] = jnp.zeros_like(cnt)

        a = ap_r[0] + ap_r[1]
        dsum = dp_r[0, :, :16] + dp_r[1, :, :16]
        hrow = lax.broadcasted_iota(jnp.int32, (16, 128), 0)
        j = lax.broadcasted_iota(jnp.int32, (16, 128), 1)
        r = jnp.where((j // 16) == hrow, 1.0, 0.0).astype(F32)
        den = _dot(dsum, r) + 1e-9
        node = a / den

        ids = b_r[0]
        g = lax.broadcasted_iota(jnp.int32, (128, bn), 0)
        oh = (jnp.broadcast_to(ids, (128, bn)) == g).astype(F32)
        gsum[...] += _dot(oh, node)
        cnt[...] += jnp.sum(oh, axis=1, keepdims=True)

        @pl.when(i == nblk - 1)
        def _():
            graph = gsum[...] / jnp.maximum(cnt[...], 1.0)
            og = jnp.maximum(_dot(graph, wr1_r[...]) + br1_r[...], 0.0)
            o_r[...] = _dot(og, wr2_r[...]) + br2_r[...]

    return pl.pallas_call(
        body,
        grid=(nblk,),
        in_specs=[
            pl.BlockSpec((2, bn, 128), lambda i: (0, i, 0)),
            pl.BlockSpec((2, bn, 128), lambda i: (0, i, 0)),
            pl.BlockSpec((1, 1, bn), lambda i: (i, 0, 0)),
            pl.BlockSpec((128, 128), lambda i: (0, 0)),
            pl.BlockSpec((1, 128), lambda i: (0, 0)),
            pl.BlockSpec((128, 1), lambda i: (0, 0)),
            pl.BlockSpec((1, 1), lambda i: (0, 0)),
        ],
        out_specs=pl.BlockSpec((128, 1), lambda i: (0, 0)),
        out_shape=jax.ShapeDtypeStruct((128, 1), F32),
        scratch_shapes=[
            pltpu.VMEM((128, 128), F32),
            pltpu.VMEM((128, 1), F32),
        ],
    )(agg_part, den_part, batch3, wr1, br1, wr2, br2)


_MESH = plsc.VectorSubcoreMesh(core_axis_name="c", subcore_axis_name="s")


def _sc_gather(tables, idxs, chunk=80):
    e = idxs[0].shape[0]
    per_w = e // NW
    n_chunks = per_w // chunk
    t_count = len(tables)

    scratch = []
    for _ in range(t_count):
        scratch.append(pltpu.VMEM((per_w,), jnp.int32))
        scratch.append(pltpu.VMEM((chunk, 128), F32))
        scratch.append(pltpu.VMEM((chunk, 128), F32))
        scratch.append(pltpu.SemaphoreType.DMA)
        scratch.append(pltpu.SemaphoreType.DMA)
        scratch.append(pltpu.VMEM((chunk,), jnp.int32))
        scratch.append(pltpu.VMEM((chunk,), jnp.int32))

    @functools.partial(
        pl.kernel,
        out_type=[jax.ShapeDtypeStruct((e, 128), F32) for _ in range(t_count)],
        mesh=_MESH,
        scratch_types=scratch,
    )
    def k(*refs):
        tabs = refs[:t_count]
        idx_hbm = refs[t_count:2 * t_count]
        outs = refs[2 * t_count:3 * t_count]
        sc = refs[3 * t_count:]
        cid = lax.axis_index("c")
        sid = lax.axis_index("s")
        base = (cid * NS + sid) * per_w

        for t in range(t_count):
            pltpu.sync_copy(idx_hbm[t].at[pl.ds(base, per_w)], sc[7 * t])

        @pl.loop(0, n_chunks)
        def _(ci):
            off = base + ci * chunk
            for t in range(t_count):
                idx_all = sc[7 * t]
                for s in range(2):
                    buf = sc[7 * t + 1 + s]
                    sem = sc[7 * t + 3 + s]
                    ib = sc[7 * t + 5 + s]

                    @pl.when(ci % 2 == s)
                    def _(buf=buf, sem=sem, ib=ib, t=t):
                        @pl.when(ci >= 2)
                        def _():
                            pltpu.make_async_copy(
                                outs[t].at[pl.ds(off, chunk)], buf, sem
                            ).wait()
                        for o in range(0, chunk, 16):
                            ib[pl.ds(o, 16)] = (
                                idx_all[pl.ds(ci * chunk + o, 16)])
                        pltpu.async_copy(tabs[t].at[ib], buf, sem).wait()
                        pltpu.async_copy(
                            buf, outs[t].at[pl.ds(off, chunk)], sem)

        for t in range(t_count):
            for s in range(2):
                buf = sc[7 * t + 1 + s]
                sem = sc[7 * t + 3 + s]
                pltpu.make_async_copy(
                    outs[t].at[pl.ds(base, chunk)], buf, sem).wait()

    res = k(*tables, *idxs)
    if not isinstance(res, (list, tuple)):
        res = [res]
    return list(res)


def _sc_scatter_add(values, idx, n, chunk=80):
    e, width = values.shape
    e2 = e // NC
    per_w = e2 // NS
    n_chunks = per_w // chunk
    npad = 10240
    rows_per_sub = npad // NS
    zrows = rows_per_sub // 5

    @functools.partial(
        pl.kernel,
        out_type=jax.ShapeDtypeStruct((NC, npad, width), F32),
        mesh=_MESH,
        scratch_types=[
            pltpu.VMEM((chunk,), jnp.int32),
            pltpu.VMEM((chunk, width), F32),
            pltpu.VMEM((zrows, width), F32),
            pltpu.VMEM_SHARED((npad, width), F32),
        ],
    )
    def k(val_hbm, idx_hbm, out_hbm, idxbuf, valbuf, zbuf, acc):
        cid = lax.axis_index("c")
        sid = lax.axis_index("s")

        @pl.loop(0, zrows)
        def _(r):
            for cc in range(width // 16):
                zbuf[r, pl.ds(cc * 16, 16)] = jnp.zeros((16,), F32)

        for kk in range(5):
            pltpu.sync_copy(
                zbuf, acc.at[pl.ds(sid * rows_per_sub + kk * zrows, zrows)])
        plsc.subcore_barrier()

        base = cid * e2 + sid * per_w

        @pl.loop(0, n_chunks)
        def _(ci):
            off = base + ci * chunk
            pltpu.sync_copy(idx_hbm.at[pl.ds(off, chunk)], idxbuf)
            pltpu.sync_copy(val_hbm.at[pl.ds(off, chunk)], valbuf)
            pltpu.sync_copy(valbuf, acc.at[idxbuf], add=True)

        plsc.subcore_barrier()
        for kk in range(5):
            rs = sid * rows_per_sub + kk * zrows
            pltpu.sync_copy(acc.at[pl.ds(rs, zrows)],
                            out_hbm.at[cid].at[pl.ds(rs, zrows)])

    return k(values, idx)[:, :n, :]


def _sc_segmax(score_flat, idx, n, chunk=1000):
    e = idx.shape[0]
    per_w = e // NW
    n_chunks = per_w // chunk
    tsz = n * 8 + 16

    @functools.partial(
        pl.kernel,
        out_type=jax.ShapeDtypeStruct((NW * n * 8,), F32),
        mesh=_MESH,
        scratch_types=[
            pltpu.VMEM((chunk + 16,), jnp.int32),
            pltpu.VMEM((chunk * 16,), F32),
            pltpu.VMEM((tsz,), F32),
        ],
        compiler_params=pltpu.CompilerParams(needs_layout_passes=False),
    )
    def k(score_hbm, idx_hbm, out_hbm, idxbuf, scorebuf, table):
        cid = lax.axis_index("c")
        sid = lax.axis_index("s")
        wid = cid * NS + sid
        base = wid * per_w

        @pl.loop(0, tsz, step=16)
        def _(i):
            table[pl.ds(i, 16)] = jnp.full((16,), -3.0e38, F32)

        lanes = lax.iota(jnp.int32, 16)
        mask = lanes < 8

        @pl.loop(0, n_chunks)
        def _(ci):
            off = base + ci * chunk
            pltpu.sync_copy(score_hbm.at[pl.ds(off * 16, chunk * 16)],
                            scorebuf)
            pltpu.sync_copy(idx_hbm.at[pl.ds(off, chunk)],
                            idxbuf.at[pl.ds(0, chunk)])

            @pl.loop(0, chunk)
            def _(ei):
                d = idxbuf[pl.ds(ei, 16)][0]
                sv = scorebuf[pl.ds(ei * 16, 16)]
                iv = d * 8 + lanes
                cur = plsc.load_gather(table, [iv], mask=mask)
                plsc.store_scatter(table, [iv],
                                   jnp.maximum(cur, sv), mask=mask)

        pltpu.sync_copy(table.at[pl.ds(0, n * 8)],
                        out_hbm.at[pl.ds(wid * n * 8, n * 8)])

    return k(score_flat, idx)


def _sc_den_exp(score_flat, smax_pad, idx, n, chunk=80):
    e = idx.shape[0]
    e2 = e // NC
    per_w = e2 // NS
    n_chunks = per_w // chunk
    npad = 10240
    rows_per_sub = npad // NS
    zrows = rows_per_sub // 5

    @functools.partial(
        pl.kernel,
        out_type=[
            jax.ShapeDtypeStruct((NC, npad, 128), F32),
            jax.ShapeDtypeStruct((e * 16,), F32),
        ],
        mesh=_MESH,
        scratch_types=[
            pltpu.VMEM((chunk,), jnp.int32),
            pltpu.VMEM((chunk * 16,), F32),
            pltpu.VMEM((chunk, 128), F32),
            pltpu.VMEM((chunk, 128), F32),
            pltpu.VMEM((chunk * 16,), F32),
            pltpu.VMEM((zrows, 128), F32),
            pltpu.VMEM_SHARED((npad, 128), F32),
        ],
    )
    def k(score_hbm, smax_hbm, idx_hbm, out_hbm, ex_hbm,
          idxbuf, sbuf, smbuf, vbuf, exbuf, zbuf, acc):
        cid = lax.axis_index("c")
        sid = lax.axis_index("s")

        @pl.loop(0, zrows)
        def _(r):
            for cc in range(8):
                zbuf[r, pl.ds(cc * 16, 16)] = jnp.zeros((16,), F32)

        for kk in range(5):
            pltpu.sync_copy(
                zbuf, acc.at[pl.ds(sid * rows_per_sub + kk * zrows, zrows)])
        plsc.subcore_barrier()

        @pl.loop(0, chunk)
        def _(r):
            for cc in range(1, 8):
                vbuf[r, pl.ds(cc * 16, 16)] = jnp.zeros((16,), F32)

        base = cid * e2 + sid * per_w

        @pl.loop(0, n_chunks)
        def _(ci):
            off = base + ci * chunk
            pltpu.sync_copy(idx_hbm.at[pl.ds(off, chunk)], idxbuf)
            pltpu.sync_copy(score_hbm.at[pl.ds(off * 16, chunk * 16)], sbuf)
            pltpu.sync_copy(smax_hbm.at[idxbuf], smbuf)

            @pl.loop(0, chunk)
            def _(ei):
                s = sbuf[pl.ds(ei * 16, 16)]
                m = smbuf[ei, pl.ds(0, 16)]
                ex = jnp.exp(s - m)
                vbuf[ei, pl.ds(0, 16)] = ex
                exbuf[pl.ds(ei * 16, 16)] = ex

            pltpu.sync_copy(vbuf, acc.at[idxbuf], add=True)
            pltpu.sync_copy(exbuf, ex_hbm.at[pl.ds(off * 16, chunk * 16)])

        plsc.subcore_barrier()
        for kk in range(5):
            rs = sid * rows_per_sub + kk * zrows
            pltpu.sync_copy(acc.at[pl.ds(rs, zrows)],
                            out_hbm.at[cid].at[pl.ds(rs, zrows)])

    den, ex_flat = k(score_flat, smax_pad, idx)
    return den[:, :n, :], ex_flat


def _tc_wv(ex_pk, vs3, be=3200):
    ep8 = ex_pk.shape[0]
    bp = be // 8

    def body(p_r, vs_r, o_r):
        p = p_r[...]
        a = lax.broadcasted_iota(jnp.int32, (128, 128), 0)
        j = lax.broadcasted_iota(jnp.int32, (128, 128), 1)
        parts = []
        for g in range(8):
            m = (a == g * 16 + j // 16).astype(F32)
            parts.append(_dot(p, m)[:, None, :])
        exb = jnp.concatenate(parts, axis=1)
        o_r[...] = vs_r[...] * exb

    return pl.pallas_call(
        body,
        grid=(ep8 // bp,),
        in_specs=[
            pl.BlockSpec((bp, 128), lambda i: (i, 0)),
            pl.BlockSpec((bp, 8, 128), lambda i: (i, 0, 0)),
        ],
        out_specs=pl.BlockSpec((bp, 8, 128), lambda i: (i, 0, 0)),
        out_shape=jax.ShapeDtypeStruct((ep8, 8, 128), F32),
    )(ex_pk, vs3)


def kernel(x, edge_index, dist_rbf, eig_pe, edge_attr, batch,
           W_phi1, b_phi1, W_phi2, b_phi2, W_edge, W_rho, b_rho,
           Wq, Wk, Wv, We, Wr1, br1, Wr2, br2):
    n = x.shape[0]
    e = edge_index.shape[1]
    src = edge_index[0]
    dst = edge_index[1]
    b_phi1 = b_phi1.reshape(1, -1)
    b_phi2 = b_phi2.reshape(1, -1)
    b_rho = b_rho.reshape(1, -1)
    br1 = br1.reshape(1, -1)
    br2 = br2.reshape(1, 1)
    batch3 = batch.reshape(n // 1000, 1, 1000)

    h = _tc_signnet_h(eig_pe, W_phi1, b_phi1, W_phi2, b_phi2)
    (hs,) = _sc_gather([h], [src], chunk=80)
    msg = _tc_msg(hs, edge_attr, W_edge)
    m_part = _sc_scatter_add(msg, dst, n)
    q, k, v = _tc_qkv(x, h, m_part, W_rho, b_rho, Wq, Wk, Wv)
    qd, ks, vs = _sc_gather([q, k, v], [dst, src, src], chunk=80)
    score = _tc_score(qd, ks, dist_rbf, We)
    score_flat = score.reshape(-1)
    smax_part = _sc_segmax(score_flat, dst, n)
    smax_pad = _tc_smax_merge(
        smax_part.reshape(NW, n // 16, 128)).reshape(n, 128)
    den_part, ex_flat = _sc_den_exp(score_flat, smax_pad, dst, n)
    wv = _tc_wv(ex_flat.reshape(e // 8, 128),
                vs.reshape(e // 8, 8, 128)).reshape(e, 128)
    agg_part = _sc_scatter_add(wv, dst, n)
    out = _tc_final(agg_part, den_part, batch3, Wr1, br1, Wr2, br2)
    return out


# speedup vs baseline: 15.0863x; 1.0932x over previous
import functools

import jax
import jax.numpy as jnp
from jax import lax
from jax.experimental import pallas as pl
from jax.experimental.pallas import tpu as pltpu
from jax.experimental.pallas import tpu_sc as plsc

F32 = jnp.float32
HI = jax.lax.Precision.HIGHEST

NC = 2
NS = 16
NW = NC * NS


def _dot(a, b):
    return jax.lax.dot(a, b, precision=HI, preferred_element_type=F32)


def _tc_signnet_h(pe, w1, b1, w2, b2):
    n = pe.shape[0]

    def body(pe_r, w1_r, b1_r, w2_r, b2_r, o_r):
        def phi(z):
            h1 = jnp.maximum(_dot(z, w1_r[...]) + b1_r[...], 0.0)
            return jnp.maximum(_dot(h1, w2_r[...]) + b2_r[...], 0.0)
        z = pe_r[...]
        o_r[...] = phi(z) + phi(-z)

    return pl.pallas_call(
        body,
        out_shape=jax.ShapeDtypeStruct((n, 128), F32),
    )(pe, w1, b1, w2, b2)


def _tc_msg(hs, edge_attr, w_edge, be=3200):
    e = hs.shape[0]

    def body(hs_r, ea_r, w_r, o_r):
        er = jnp.maximum(_dot(ea_r[...], w_r[...]), 0.0)
        o_r[...] = jnp.maximum(hs_r[...] + er, 0.0)

    return pl.pallas_call(
        body,
        grid=(e // be,),
        in_specs=[
            pl.BlockSpec((be, 128), lambda i: (i, 0)),
            pl.BlockSpec((be, 16), lambda i: (i, 0)),
            pl.BlockSpec((16, 128), lambda i: (0, 0)),
        ],
        out_specs=pl.BlockSpec((be, 128), lambda i: (i, 0)),
        out_shape=jax.ShapeDtypeStruct((e, 128), F32),
    )(hs, edge_attr, w_edge)


def _tc_qkv(x, h, m_part, w_rho, b_rho, wq, wk, wv, bn=2000):
    n = x.shape[0]

    def body(x_r, h_r, mp_r, wr_r, br_r, wq_r, wk_r, wv_r, q_r, k_r, v_r):
        hm = h_r[...] + mp_r[0] + mp_r[1]
        pos = _dot(hm, wr_r[...]) + br_r[...]
        hn = x_r[...] + pos
        q_r[...] = _dot(hn, wq_r[...])
        k_r[...] = _dot(hn, wk_r[...])
        v_r[...] = _dot(hn, wv_r[...])

    spec = pl.BlockSpec((bn, 128), lambda i: (i, 0))
    wspec = pl.BlockSpec((128, 128), lambda i: (0, 0))
    return pl.pallas_call(
        body,
        grid=(n // bn,),
        in_specs=[
            spec,
            spec,
            pl.BlockSpec((2, bn, 128), lambda i: (0, i, 0)),
            wspec,
            pl.BlockSpec((1, 128), lambda i: (0, 0)),
            wspec, wspec, wspec,
        ],
        out_specs=[spec, spec, spec],
        out_shape=[jax.ShapeDtypeStruct((n, 128), F32)] * 3,
    )(x, h, m_part, w_rho, b_rho, wq, wk, wv)


def _tc_score(qd, ks, dist_rbf, we, be=3200):
    e = qd.shape[0]

    def body(qd_r, ks_r, rbf_r, we_r, o_r):
        eb = _dot(rbf_r[...], we_r[...])
        p = qd_r[...] * (ks_r[...] + eb)
        j = lax.broadcasted_iota(jnp.int32, (128, 16), 0)
        hcol = lax.broadcasted_iota(jnp.int32, (128, 16), 1)
        t = jnp.where((j // 16) == hcol, 0.25, 0.0).astype(F32)
        o_r[...] = _dot(p, t)

    return pl.pallas_call(
        body,
        grid=(e // be,),
        in_specs=[
            pl.BlockSpec((be, 128), lambda i: (i, 0)),
            pl.BlockSpec((be, 128), lambda i: (i, 0)),
            pl.BlockSpec((be, 16), lambda i: (i, 0)),
            pl.BlockSpec((16, 128), lambda i: (0, 0)),
        ],
        out_specs=pl.BlockSpec((be, 16), lambda i: (i, 0)),
        out_shape=jax.ShapeDtypeStruct((e, 16), F32),
    )(qd, ks, dist_rbf, we)


def _tc_smax_merge(smax_part, bm=625):
    npk = smax_part.shape[1]

    def body(sp_r, o_r):
        pk = jnp.max(sp_r[...], axis=0)
        z = jnp.zeros((pk.shape[0], 120), F32)
        parts = [
            jnp.concatenate([pk[:, g * 8:(g + 1) * 8], z], axis=1)[:, None, :]
            for g in range(16)
        ]
        o_r[...] = jnp.concatenate(parts, axis=1)

    return pl.pallas_call(
        body,
        grid=(npk // bm,),
        in_specs=[pl.BlockSpec((NW, bm, 128), lambda i: (0, i, 0))],
        out_specs=pl.BlockSpec((bm, 16, 128), lambda i: (i, 0, 0)),
        out_shape=jax.ShapeDtypeStruct((npk, 16, 128), F32),
    )(smax_part)


def _tc_final(agg_part, den_part, batch3, wr1, br1, wr2, br2, bn=1000):
    n = agg_part.shape[1]
    nblk = n // bn

    def body(ap_r, dp_r, b_r, wr1_r, br1_r, wr2_r, br2_r, o_r, gsum, cnt):
        i = pl.program_id(0)

        @pl.when(i == 0)
        def _():
            gsum[...] = jnp.zeros_like(gsum)
            cnt[...] = jnp.zeros_like(cnt)

        a = ap_r[0] + ap_r[1]
        dsum = dp_r[0, :, :16] + dp_r[1, :, :16]
        hrow = lax.broadcasted_iota(jnp.int32, (16, 128), 0)
        j = lax.broadcasted_iota(jnp.int32, (16, 128), 1)
        r = jnp.where((j // 16) == hrow, 1.0, 0.0).astype(F32)
        den = _dot(dsum, r) + 1e-9
        node = a / den

        ids = b_r[0]
        g = lax.broadcasted_iota(jnp.int32, (128, bn), 0)
        oh = (jnp.broadcast_to(ids, (128, bn)) == g).astype(F32)
        gsum[...] += _dot(oh, node)
        cnt[...] += jnp.sum(oh, axis=1, keepdims=True)

        @pl.when(i == nblk - 1)
        def _():
            graph = gsum[...] / jnp.maximum(cnt[...], 1.0)
            og = jnp.maximum(_dot(graph, wr1_r[...]) + br1_r[...], 0.0)
            o_r[...] = _dot(og, wr2_r[...]) + br2_r[...]

    return pl.pallas_call(
        body,
        grid=(nblk,),
        in_specs=[
            pl.BlockSpec((2, bn, 128), lambda i: (0, i, 0)),
            pl.BlockSpec((2, bn, 128), lambda i: (0, i, 0)),
            pl.BlockSpec((1, 1, bn), lambda i: (i, 0, 0)),
            pl.BlockSpec((128, 128), lambda i: (0, 0)),
            pl.BlockSpec((1, 128), lambda i: (0, 0)),
            pl.BlockSpec((128, 1), lambda i: (0, 0)),
            pl.BlockSpec((1, 1), lambda i: (0, 0)),
        ],
        out_specs=pl.BlockSpec((128, 1), lambda i: (0, 0)),
        out_shape=jax.ShapeDtypeStruct((128, 1), F32),
        scratch_shapes=[
            pltpu.VMEM((128, 128), F32),
            pltpu.VMEM((128, 1), F32),
        ],
    )(agg_part, den_part, batch3, wr1, br1, wr2, br2)


_MESH = plsc.VectorSubcoreMesh(core_axis_name="c", subcore_axis_name="s")


def _sc_gather(tables, idxs, chunk=80):
    e = idxs[0].shape[0]
    per_w = e // NW
    n_chunks = per_w // chunk
    t_count = len(tables)

    scratch = []
    for _ in range(t_count):
        scratch.append(pltpu.VMEM((per_w,), jnp.int32))
        scratch.append(pltpu.VMEM((chunk, 128), F32))
        scratch.append(pltpu.VMEM((chunk, 128), F32))
        scratch.append(pltpu.SemaphoreType.DMA)
        scratch.append(pltpu.SemaphoreType.DMA)
        scratch.append(pltpu.VMEM((chunk,), jnp.int32))
        scratch.append(pltpu.VMEM((chunk,), jnp.int32))

    @functools.partial(
        pl.kernel,
        out_type=[jax.ShapeDtypeStruct((e, 128), F32) for _ in range(t_count)],
        mesh=_MESH,
        scratch_types=scratch,
    )
    def k(*refs):
        tabs = refs[:t_count]
        idx_hbm = refs[t_count:2 * t_count]
        outs = refs[2 * t_count:3 * t_count]
        sc = refs[3 * t_count:]
        cid = lax.axis_index("c")
        sid = lax.axis_index("s")
        base = (cid * NS + sid) * per_w

        for t in range(t_count):
            pltpu.sync_copy(idx_hbm[t].at[pl.ds(base, per_w)], sc[7 * t])

        @pl.loop(0, n_chunks)
        def _(ci):
            off = base + ci * chunk
            for t in range(t_count):
                idx_all = sc[7 * t]
                for s in range(2):
                    buf = sc[7 * t + 1 + s]
                    sem = sc[7 * t + 3 + s]
                    ib = sc[7 * t + 5 + s]

                    @pl.when(ci % 2 == s)
                    def _(buf=buf, sem=sem, ib=ib, t=t):
                        @pl.when(ci >= 2)
                        def _():
                            pltpu.make_async_copy(
                                outs[t].at[pl.ds(off, chunk)], buf, sem
                            ).wait()
                        for o in range(0, chunk, 16):
                            ib[pl.ds(o, 16)] = (
                                idx_all[pl.ds(ci * chunk + o, 16)])
                        pltpu.async_copy(tabs[t].at[ib], buf, sem).wait()
                        pltpu.async_copy(
                            buf, outs[t].at[pl.ds(off, chunk)], sem)

        for t in range(t_count):
            for s in range(2):
                buf = sc[7 * t + 1 + s]
                sem = sc[7 * t + 3 + s]
                pltpu.make_async_copy(
                    outs[t].at[pl.ds(base, chunk)], buf, sem).wait()

    res = k(*tables, *idxs)
    if not isinstance(res, (list, tuple)):
        res = [res]
    return list(res)


def _sc_scatter_add(values, idx, n, chunk=80):
    e, width = values.shape
    e2 = e // NC
    per_w = e2 // NS
    n_chunks = per_w // chunk
    npad = 10240
    rows_per_sub = npad // NS
    zrows = rows_per_sub // 5

    @functools.partial(
        pl.kernel,
        out_type=jax.ShapeDtypeStruct((NC, npad, width), F32),
        mesh=_MESH,
        scratch_types=[
            pltpu.VMEM((per_w,), jnp.int32),
            pltpu.VMEM((chunk,), jnp.int32),
            pltpu.VMEM((chunk,), jnp.int32),
            pltpu.VMEM((chunk, width), F32),
            pltpu.VMEM((chunk, width), F32),
            pltpu.VMEM((zrows, width), F32),
            pltpu.VMEM_SHARED((npad, width), F32),
            pltpu.SemaphoreType.DMA,
            pltpu.SemaphoreType.DMA,
            pltpu.SemaphoreType.DMA,
            pltpu.SemaphoreType.DMA,
        ],
    )
    def k(val_hbm, idx_hbm, out_hbm, idx_all, ib0, ib1, vb0, vb1, zbuf,
          acc, sl0, sl1, ss0, ss1):
        cid = lax.axis_index("c")
        sid = lax.axis_index("s")
        ibs, vbs, sls, sss = (ib0, ib1), (vb0, vb1), (sl0, sl1), (ss0, ss1)

        @pl.loop(0, zrows)
        def _(r):
            for cc in range(width // 16):
                zbuf[r, pl.ds(cc * 16, 16)] = jnp.zeros((16,), F32)

        for kk in range(5):
            pltpu.sync_copy(
                zbuf, acc.at[pl.ds(sid * rows_per_sub + kk * zrows, zrows)])
        plsc.subcore_barrier()

        base = cid * e2 + sid * per_w
        pltpu.sync_copy(idx_hbm.at[pl.ds(base, per_w)], idx_all)
        pltpu.async_copy(val_hbm.at[pl.ds(base, chunk)], vb0, sl0)

        @pl.loop(0, n_chunks)
        def _(ci):
            off = base + ci * chunk
            for s in range(2):
                s2 = 1 - s

                @pl.when(ci % 2 == s)
                def _(s=s, s2=s2):
                    @pl.when(ci >= 1)
                    def _():
                        pltpu.make_async_copy(
                            val_hbm.at[pl.ds(off, chunk)], vbs[s2], sss[s2]
                        ).wait()

                    @pl.when(ci + 1 < n_chunks)
                    def _():
                        pltpu.async_copy(
                            val_hbm.at[pl.ds(off + chunk, chunk)],
                            vbs[s2], sls[s2])

                    pltpu.make_async_copy(
                        val_hbm.at[pl.ds(off, chunk)], vbs[s], sls[s]
                    ).wait()
                    for o in range(0, chunk, 16):
                        ibs[s][pl.ds(o, 16)] = (
                            idx_all[pl.ds(ci * chunk + o, 16)])
                    pltpu.async_copy(
                        vbs[s], acc.at[ibs[s]], sss[s], add=True)

        ls = (n_chunks - 1) % 2
        pltpu.make_async_copy(
            val_hbm.at[pl.ds(base, chunk)], vbs[ls], sss[ls]).wait()

        plsc.subcore_barrier()
        for kk in range(5):
            rs = sid * rows_per_sub + kk * zrows
            pltpu.sync_copy(acc.at[pl.ds(rs, zrows)],
                            out_hbm.at[cid].at[pl.ds(rs, zrows)])

    return k(values, idx)[:, :n, :]


def _sc_segmax(score_flat, idx, n, chunk=1000):
    e = idx.shape[0]
    per_w = e // NW
    n_chunks = per_w // chunk
    tsz = n * 8 + 16

    @functools.partial(
        pl.kernel,
        out_type=jax.ShapeDtypeStruct((NW * n * 8,), F32),
        mesh=_MESH,
        scratch_types=[
            pltpu.VMEM((chunk + 16,), jnp.int32),
            pltpu.VMEM((chunk * 16,), F32),
            pltpu.VMEM((tsz,), F32),
        ],
        compiler_params=pltpu.CompilerParams(needs_layout_passes=False),
    )
    def k(score_hbm, idx_hbm, out_hbm, idxbuf, scorebuf, table):
        cid = lax.axis_index("c")
        sid = lax.axis_index("s")
        wid = cid * NS + sid
        base = wid * per_w

        @pl.loop(0, tsz, step=16)
        def _(i):
            table[pl.ds(i, 16)] = jnp.full((16,), -3.0e38, F32)

        lanes = lax.iota(jnp.int32, 16)
        mask = lanes < 8

        @pl.loop(0, n_chunks)
        def _(ci):
            off = base + ci * chunk
            pltpu.sync_copy(score_hbm.at[pl.ds(off * 16, chunk * 16)],
                            scorebuf)
            pltpu.sync_copy(idx_hbm.at[pl.ds(off, chunk)],
                            idxbuf.at[pl.ds(0, chunk)])

            @pl.loop(0, chunk)
            def _(ei):
                d = idxbuf[pl.ds(ei, 16)][0]
                sv = scorebuf[pl.ds(ei * 16, 16)]
                iv = d * 8 + lanes
                cur = plsc.load_gather(table, [iv], mask=mask)
                plsc.store_scatter(table, [iv],
                                   jnp.maximum(cur, sv), mask=mask)

        pltpu.sync_copy(table.at[pl.ds(0, n * 8)],
                        out_hbm.at[pl.ds(wid * n * 8, n * 8)])

    return k(score_flat, idx)


def _sc_den_exp(score_flat, smax_pad, idx, n, chunk=80):
    e = idx.shape[0]
    e2 = e // NC
    per_w = e2 // NS
    n_chunks = per_w // chunk
    npad = 10240
    rows_per_sub = npad // NS
    zrows = rows_per_sub // 5

    @functools.partial(
        pl.kernel,
        out_type=[
            jax.ShapeDtypeStruct((NC, npad, 128), F32),
            jax.ShapeDtypeStruct((e * 16,), F32),
        ],
        mesh=_MESH,
        scratch_types=[
            pltpu.VMEM((chunk,), jnp.int32),
            pltpu.VMEM((chunk * 16,), F32),
            pltpu.VMEM((chunk, 128), F32),
            pltpu.VMEM((chunk, 128), F32),
            pltpu.VMEM((chunk * 16,), F32),
            pltpu.VMEM((zrows, 128), F32),
            pltpu.VMEM_SHARED((npad, 128), F32),
        ],
    )
    def k(score_hbm, smax_hbm, idx_hbm, out_hbm, ex_hbm,
          idxbuf, sbuf, smbuf, vbuf, exbuf, zbuf, acc):
        cid = lax.axis_index("c")
        sid = lax.axis_index("s")

        @pl.loop(0, zrows)
        def _(r):
            for cc in range(8):
                zbuf[r, pl.ds(cc * 16, 16)] = jnp.zeros((16,), F32)

        for kk in range(5):
            pltpu.sync_copy(
                zbuf, acc.at[pl.ds(sid * rows_per_sub + kk * zrows, zrows)])
        plsc.subcore_barrier()

        @pl.loop(0, chunk)
        def _(r):
            for cc in range(1, 8):
                vbuf[r, pl.ds(cc * 16, 16)] = jnp.zeros((16,), F32)

        base = cid * e2 + sid * per_w

        @pl.loop(0, n_chunks)
        def _(ci):
            off = base + ci * chunk
            pltpu.sync_copy(idx_hbm.at[pl.ds(off, chunk)], idxbuf)
            pltpu.sync_copy(score_hbm.at[pl.ds(off * 16, chunk * 16)], sbuf)
            pltpu.sync_copy(smax_hbm.at[idxbuf], smbuf)

            @pl.loop(0, chunk)
            def _(ei):
                s = sbuf[pl.ds(ei * 16, 16)]
                m = smbuf[ei, pl.ds(0, 16)]
                ex = jnp.exp(s - m)
                vbuf[ei, pl.ds(0, 16)] = ex
                exbuf[pl.ds(ei * 16, 16)] = ex

            pltpu.sync_copy(vbuf, acc.at[idxbuf], add=True)
            pltpu.sync_copy(exbuf, ex_hbm.at[pl.ds(off * 16, chunk * 16)])

        plsc.subcore_barrier()
        for kk in range(5):
            rs = sid * rows_per_sub + kk * zrows
            pltpu.sync_copy(acc.at[pl.ds(rs, zrows)],
                            out_hbm.at[cid].at[pl.ds(rs, zrows)])

    den, ex_flat = k(score_flat, smax_pad, idx)
    return den[:, :n, :], ex_flat


def _tc_wv(ex_pk, vs3, be=3200):
    ep8 = ex_pk.shape[0]
    bp = be // 8

    def body(p_r, vs_r, o_r):
        p = p_r[...]
        a = lax.broadcasted_iota(jnp.int32, (128, 128), 0)
        j = lax.broadcasted_iota(jnp.int32, (128, 128), 1)
        parts = []
        for g in range(8):
            m = (a == g * 16 + j // 16).astype(F32)
            parts.append(_dot(p, m)[:, None, :])
        exb = jnp.concatenate(parts, axis=1)
        o_r[...] = vs_r[...] * exb

    return pl.pallas_call(
        body,
        grid=(ep8 // bp,),
        in_specs=[
            pl.BlockSpec((bp, 128), lambda i: (i, 0)),
            pl.BlockSpec((bp, 8, 128), lambda i: (i, 0, 0)),
        ],
        out_specs=pl.BlockSpec((bp, 8, 128), lambda i: (i, 0, 0)),
        out_shape=jax.ShapeDtypeStruct((ep8, 8, 128), F32),
    )(ex_pk, vs3)


def kernel(x, edge_index, dist_rbf, eig_pe, edge_attr, batch,
           W_phi1, b_phi1, W_phi2, b_phi2, W_edge, W_rho, b_rho,
           Wq, Wk, Wv, We, Wr1, br1, Wr2, br2):
    n = x.shape[0]
    e = edge_index.shape[1]
    src = edge_index[0]
    dst = edge_index[1]
    b_phi1 = b_phi1.reshape(1, -1)
    b_phi2 = b_phi2.reshape(1, -1)
    b_rho = b_rho.reshape(1, -1)
    br1 = br1.reshape(1, -1)
    br2 = br2.reshape(1, 1)
    batch3 = batch.reshape(n // 1000, 1, 1000)

    h = _tc_signnet_h(eig_pe, W_phi1, b_phi1, W_phi2, b_phi2)
    (hs,) = _sc_gather([h], [src], chunk=80)
    msg = _tc_msg(hs, edge_attr, W_edge)
    m_part = _sc_scatter_add(msg, dst, n)
    q, k, v = _tc_qkv(x, h, m_part, W_rho, b_rho, Wq, Wk, Wv)
    qd, ks, vs = _sc_gather([q, k, v], [dst, src, src], chunk=80)
    score = _tc_score(qd, ks, dist_rbf, We)
    score_flat = score.reshape(-1)
    smax_part = _sc_segmax(score_flat, dst, n)
    smax_pad = _tc_smax_merge(
        smax_part.reshape(NW, n // 16, 128)).reshape(n, 128)
    den_part, ex_flat = _sc_den_exp(score_flat, smax_pad, dst, n)
    wv = _tc_wv(ex_flat.reshape(e // 8, 128),
                vs.reshape(e // 8, 8, 128)).reshape(e, 128)
    agg_part = _sc_scatter_add(wv, dst, n)
    out = _tc_final(agg_part, den_part, batch3, Wr1, br1, Wr2, br2)
    return out


# speedup vs baseline: 16.5817x; 1.0991x over previous
import functools

import jax
import jax.numpy as jnp
from jax import lax
from jax.experimental import pallas as pl
from jax.experimental.pallas import tpu as pltpu
from jax.experimental.pallas import tpu_sc as plsc

F32 = jnp.float32
HI = jax.lax.Precision.HIGHEST

NC = 2
NS = 16
NW = NC * NS


def _dot(a, b):
    return jax.lax.dot(a, b, precision=HI, preferred_element_type=F32)


def _tc_signnet_h(pe, w1, b1, w2, b2):
    n = pe.shape[0]

    def body(pe_r, w1_r, b1_r, w2_r, b2_r, o_r):
        def phi(z):
            h1 = jnp.maximum(_dot(z, w1_r[...]) + b1_r[...], 0.0)
            return jnp.maximum(_dot(h1, w2_r[...]) + b2_r[...], 0.0)
        z = pe_r[...]
        o_r[...] = phi(z) + phi(-z)

    return pl.pallas_call(
        body,
        out_shape=jax.ShapeDtypeStruct((n, 128), F32),
    )(pe, w1, b1, w2, b2)


def _tc_msg(hs, edge_attr, w_edge, be=3200):
    e = hs.shape[0]

    def body(hs_r, ea_r, w_r, o_r):
        er = jnp.maximum(_dot(ea_r[...], w_r[...]), 0.0)
        o_r[...] = jnp.maximum(hs_r[...] + er, 0.0)

    return pl.pallas_call(
        body,
        grid=(e // be,),
        in_specs=[
            pl.BlockSpec((be, 128), lambda i: (i, 0)),
            pl.BlockSpec((be, 16), lambda i: (i, 0)),
            pl.BlockSpec((16, 128), lambda i: (0, 0)),
        ],
        out_specs=pl.BlockSpec((be, 128), lambda i: (i, 0)),
        out_shape=jax.ShapeDtypeStruct((e, 128), F32),
    )(hs, edge_attr, w_edge)


def _tc_qkv(x, h, m_part, w_rho, b_rho, wq, wk, wv, bn=2000):
    n = x.shape[0]

    def body(x_r, h_r, mp_r, wr_r, br_r, wq_r, wk_r, wv_r, q_r, k_r, v_r):
        hm = h_r[...] + mp_r[0] + mp_r[1]
        pos = _dot(hm, wr_r[...]) + br_r[...]
        hn = x_r[...] + pos
        q_r[...] = _dot(hn, wq_r[...])
        k_r[...] = _dot(hn, wk_r[...])
        v_r[...] = _dot(hn, wv_r[...])

    spec = pl.BlockSpec((bn, 128), lambda i: (i, 0))
    wspec = pl.BlockSpec((128, 128), lambda i: (0, 0))
    return pl.pallas_call(
        body,
        grid=(n // bn,),
        in_specs=[
            spec,
            spec,
            pl.BlockSpec((2, bn, 128), lambda i: (0, i, 0)),
            wspec,
            pl.BlockSpec((1, 128), lambda i: (0, 0)),
            wspec, wspec, wspec,
        ],
        out_specs=[spec, spec, spec],
        out_shape=[jax.ShapeDtypeStruct((n, 128), F32)] * 3,
    )(x, h, m_part, w_rho, b_rho, wq, wk, wv)


def _tc_score(qd, ks, dist_rbf, we, be=3200):
    e = qd.shape[0]

    def body(qd_r, ks_r, rbf_r, we_r, o_r):
        eb = _dot(rbf_r[...], we_r[...])
        p = qd_r[...] * (ks_r[...] + eb)
        j = lax.broadcasted_iota(jnp.int32, (128, 16), 0)
        hcol = lax.broadcasted_iota(jnp.int32, (128, 16), 1)
        t = jnp.where((j // 16) == hcol, 0.25, 0.0).astype(F32)
        o_r[...] = _dot(p, t)

    return pl.pallas_call(
        body,
        grid=(e // be,),
        in_specs=[
            pl.BlockSpec((be, 128), lambda i: (i, 0)),
            pl.BlockSpec((be, 128), lambda i: (i, 0)),
            pl.BlockSpec((be, 16), lambda i: (i, 0)),
            pl.BlockSpec((16, 128), lambda i: (0, 0)),
        ],
        out_specs=pl.BlockSpec((be, 16), lambda i: (i, 0)),
        out_shape=jax.ShapeDtypeStruct((e, 16), F32),
    )(qd, ks, dist_rbf, we)


def _tc_smax_merge(smax_part, bm=625):
    npk = smax_part.shape[1]

    def body(sp_r, o_r):
        pk = jnp.max(sp_r[...], axis=0)
        z = jnp.zeros((pk.shape[0], 120), F32)
        parts = [
            jnp.concatenate([pk[:, g * 8:(g + 1) * 8], z], axis=1)[:, None, :]
            for g in range(16)
        ]
        o_r[...] = jnp.concatenate(parts, axis=1)

    return pl.pallas_call(
        body,
        grid=(npk // bm,),
        in_specs=[pl.BlockSpec((NW, bm, 128), lambda i: (0, i, 0))],
        out_specs=pl.BlockSpec((bm, 16, 128), lambda i: (i, 0, 0)),
        out_shape=jax.ShapeDtypeStruct((npk, 16, 128), F32),
    )(smax_part)


def _tc_final(agg_part, den_part, batch3, wr1, br1, wr2, br2, bn=1000):
    n = agg_part.shape[1]
    nblk = n // bn

    def body(ap_r, dp_r, b_r, wr1_r, br1_r, wr2_r, br2_r, o_r, gsum, cnt):
        i = pl.program_id(0)

        @pl.when(i == 0)
        def _():
            gsum[...] = jnp.zeros_like(gsum)
            cnt[...] = jnp.zeros_like(cnt)

        a = ap_r[0] + ap_r[1]
        dsum = dp_r[0, :, :16] + dp_r[1, :, :16]
        hrow = lax.broadcasted_iota(jnp.int32, (16, 128), 0)
        j = lax.broadcasted_iota(jnp.int32, (16, 128), 1)
        r = jnp.where((j // 16) == hrow, 1.0, 0.0).astype(F32)
        den = _dot(dsum, r) + 1e-9
        node = a / den

        ids = b_r[0]
        g = lax.broadcasted_iota(jnp.int32, (128, bn), 0)
        oh = (jnp.broadcast_to(ids, (128, bn)) == g).astype(F32)
        gsum[...] += _dot(oh, node)
        cnt[...] += jnp.sum(oh, axis=1, keepdims=True)

        @pl.when(i == nblk - 1)
        def _():
            graph = gsum[...] / jnp.maximum(cnt[...], 1.0)
            og = jnp.maximum(_dot(graph, wr1_r[...]) + br1_r[...], 0.0)
            o_r[...] = _dot(og, wr2_r[...]) + br2_r[...]

    return pl.pallas_call(
        body,
        grid=(nblk,),
        in_specs=[
            pl.BlockSpec((2, bn, 128), lambda i: (0, i, 0)),
            pl.BlockSpec((2, bn, 128), lambda i: (0, i, 0)),
            pl.BlockSpec((1, 1, bn), lambda i: (i, 0, 0)),
            pl.BlockSpec((128, 128), lambda i: (0, 0)),
            pl.BlockSpec((1, 128), lambda i: (0, 0)),
            pl.BlockSpec((128, 1), lambda i: (0, 0)),
            pl.BlockSpec((1, 1), lambda i: (0, 0)),
        ],
        out_specs=pl.BlockSpec((128, 1), lambda i: (0, 0)),
        out_shape=jax.ShapeDtypeStruct((128, 1), F32),
        scratch_shapes=[
            pltpu.VMEM((128, 128), F32),
            pltpu.VMEM((128, 1), F32),
        ],
    )(agg_part, den_part, batch3, wr1, br1, wr2, br2)


_MESH = plsc.VectorSubcoreMesh(core_axis_name="c", subcore_axis_name="s")


def _sc_gather(tables, idxs, chunk=80):
    e = idxs[0].shape[0]
    per_w = e // NW
    n_chunks = per_w // chunk
    t_count = len(tables)

    scratch = []
    for _ in range(t_count):
        scratch.append(pltpu.VMEM((per_w,), jnp.int32))
        scratch.append(pltpu.VMEM((chunk, 128), F32))
        scratch.append(pltpu.VMEM((chunk, 128), F32))
        scratch.append(pltpu.SemaphoreType.DMA)
        scratch.append(pltpu.SemaphoreType.DMA)
        scratch.append(pltpu.VMEM((chunk,), jnp.int32))
        scratch.append(pltpu.VMEM((chunk,), jnp.int32))

    @functools.partial(
        pl.kernel,
        out_type=[jax.ShapeDtypeStruct((e, 128), F32) for _ in range(t_count)],
        mesh=_MESH,
        scratch_types=scratch,
    )
    def k(*refs):
        tabs = refs[:t_count]
        idx_hbm = refs[t_count:2 * t_count]
        outs = refs[2 * t_count:3 * t_count]
        sc = refs[3 * t_count:]
        cid = lax.axis_index("c")
        sid = lax.axis_index("s")
        base = (cid * NS + sid) * per_w

        for t in range(t_count):
            pltpu.sync_copy(idx_hbm[t].at[pl.ds(base, per_w)], sc[7 * t])

        @pl.loop(0, n_chunks)
        def _(ci):
            off = base + ci * chunk
            for t in range(t_count):
                idx_all = sc[7 * t]
                for s in range(2):
                    buf = sc[7 * t + 1 + s]
                    sem = sc[7 * t + 3 + s]
                    ib = sc[7 * t + 5 + s]

                    @pl.when(ci % 2 == s)
                    def _(buf=buf, sem=sem, ib=ib, t=t):
                        @pl.when(ci >= 2)
                        def _():
                            pltpu.make_async_copy(
                                outs[t].at[pl.ds(off, chunk)], buf, sem
                            ).wait()
                        for o in range(0, chunk, 16):
                            ib[pl.ds(o, 16)] = (
                                idx_all[pl.ds(ci * chunk + o, 16)])
                        pltpu.async_copy(tabs[t].at[ib], buf, sem).wait()
                        pltpu.async_copy(
                            buf, outs[t].at[pl.ds(off, chunk)], sem)

        for t in range(t_count):
            for s in range(2):
                buf = sc[7 * t + 1 + s]
                sem = sc[7 * t + 3 + s]
                pltpu.make_async_copy(
                    outs[t].at[pl.ds(base, chunk)], buf, sem).wait()

    res = k(*tables, *idxs)
    if not isinstance(res, (list, tuple)):
        res = [res]
    return list(res)


def _sc_scatter_add(values, idx, n, chunk=80):
    e, width = values.shape
    e2 = e // NC
    per_w = e2 // NS
    n_chunks = per_w // chunk
    npad = 10240
    rows_per_sub = npad // NS
    zrows = rows_per_sub // 5

    @functools.partial(
        pl.kernel,
        out_type=jax.ShapeDtypeStruct((NC, npad, width), F32),
        mesh=_MESH,
        scratch_types=[
            pltpu.VMEM((per_w,), jnp.int32),
            pltpu.VMEM((chunk,), jnp.int32),
            pltpu.VMEM((chunk,), jnp.int32),
            pltpu.VMEM((chunk, width), F32),
            pltpu.VMEM((chunk, width), F32),
            pltpu.VMEM((zrows, width), F32),
            pltpu.VMEM_SHARED((npad, width), F32),
            pltpu.SemaphoreType.DMA,
            pltpu.SemaphoreType.DMA,
            pltpu.SemaphoreType.DMA,
            pltpu.SemaphoreType.DMA,
        ],
    )
    def k(val_hbm, idx_hbm, out_hbm, idx_all, ib0, ib1, vb0, vb1, zbuf,
          acc, sl0, sl1, ss0, ss1):
        cid = lax.axis_index("c")
        sid = lax.axis_index("s")
        ibs, vbs, sls, sss = (ib0, ib1), (vb0, vb1), (sl0, sl1), (ss0, ss1)

        @pl.loop(0, zrows)
        def _(r):
            for cc in range(width // 16):
                zbuf[r, pl.ds(cc * 16, 16)] = jnp.zeros((16,), F32)

        for kk in range(5):
            pltpu.sync_copy(
                zbuf, acc.at[pl.ds(sid * rows_per_sub + kk * zrows, zrows)])
        plsc.subcore_barrier()

        base = cid * e2 + sid * per_w
        pltpu.sync_copy(idx_hbm.at[pl.ds(base, per_w)], idx_all)
        pltpu.async_copy(val_hbm.at[pl.ds(base, chunk)], vb0, sl0)

        @pl.loop(0, n_chunks)
        def _(ci):
            off = base + ci * chunk
            for s in range(2):
                s2 = 1 - s

                @pl.when(ci % 2 == s)
                def _(s=s, s2=s2):
                    @pl.when(ci >= 1)
                    def _():
                        pltpu.make_async_copy(
                            val_hbm.at[pl.ds(off, chunk)], vbs[s2], sss[s2]
                        ).wait()

                    @pl.when(ci + 1 < n_chunks)
                    def _():
                        pltpu.async_copy(
                            val_hbm.at[pl.ds(off + chunk, chunk)],
                            vbs[s2], sls[s2])

                    pltpu.make_async_copy(
                        val_hbm.at[pl.ds(off, chunk)], vbs[s], sls[s]
                    ).wait()
                    for o in range(0, chunk, 16):
                        ibs[s][pl.ds(o, 16)] = (
                            idx_all[pl.ds(ci * chunk + o, 16)])
                    pltpu.async_copy(
                        vbs[s], acc.at[ibs[s]], sss[s], add=True)

        ls = (n_chunks - 1) % 2
        pltpu.make_async_copy(
            val_hbm.at[pl.ds(base, chunk)], vbs[ls], sss[ls]).wait()

        plsc.subcore_barrier()
        for kk in range(5):
            rs = sid * rows_per_sub + kk * zrows
            pltpu.sync_copy(acc.at[pl.ds(rs, zrows)],
                            out_hbm.at[cid].at[pl.ds(rs, zrows)])

    return k(values, idx)[:, :n, :]


def _sc_segmax(score_flat, idx, n, chunk=1000):
    e = idx.shape[0]
    per_w = e // NW
    n_chunks = per_w // chunk
    tsz = n * 8 + 16

    @functools.partial(
        pl.kernel,
        out_type=jax.ShapeDtypeStruct((NW * n * 8,), F32),
        mesh=_MESH,
        scratch_types=[
            pltpu.VMEM((chunk + 16,), jnp.int32),
            pltpu.VMEM((chunk * 16,), F32),
            pltpu.VMEM((tsz,), F32),
        ],
        compiler_params=pltpu.CompilerParams(needs_layout_passes=False),
    )
    def k(score_hbm, idx_hbm, out_hbm, idxbuf, scorebuf, table):
        cid = lax.axis_index("c")
        sid = lax.axis_index("s")
        wid = cid * NS + sid
        base = wid * per_w

        @pl.loop(0, tsz, step=16)
        def _(i):
            table[pl.ds(i, 16)] = jnp.full((16,), -3.0e38, F32)

        lanes = lax.iota(jnp.int32, 16)
        mask = lanes < 8

        @pl.loop(0, n_chunks)
        def _(ci):
            off = base + ci * chunk
            pltpu.sync_copy(score_hbm.at[pl.ds(off * 16, chunk * 16)],
                            scorebuf)
            pltpu.sync_copy(idx_hbm.at[pl.ds(off, chunk)],
                            idxbuf.at[pl.ds(0, chunk)])

            @pl.loop(0, chunk)
            def _(ei):
                d = idxbuf[pl.ds(ei, 16)][0]
                sv = scorebuf[pl.ds(ei * 16, 16)]
                iv = d * 8 + lanes
                cur = plsc.load_gather(table, [iv], mask=mask)
                plsc.store_scatter(table, [iv],
                                   jnp.maximum(cur, sv), mask=mask)

        pltpu.sync_copy(table.at[pl.ds(0, n * 8)],
                        out_hbm.at[pl.ds(wid * n * 8, n * 8)])

    return k(score_flat, idx)


def _sc_den_exp(score_flat, smax_pad, idx, n, chunk=80):
    e = idx.shape[0]
    e2 = e // NC
    per_w = e2 // NS
    n_chunks = per_w // chunk
    npad = 10240
    rows_per_sub = npad // NS
    zrows = 32

    @functools.partial(
        pl.kernel,
        out_type=[
            jax.ShapeDtypeStruct((NC, npad, 128), F32),
            jax.ShapeDtypeStruct((e * 16,), F32),
        ],
        mesh=_MESH,
        scratch_types=[
            pltpu.VMEM((per_w,), jnp.int32),
            pltpu.VMEM((chunk,), jnp.int32),
            pltpu.VMEM((chunk,), jnp.int32),
            pltpu.VMEM((chunk * 16,), F32),
            pltpu.VMEM((chunk * 16,), F32),
            pltpu.VMEM((chunk, 128), F32),
            pltpu.VMEM((chunk, 128), F32),
            pltpu.VMEM((chunk, 128), F32),
            pltpu.VMEM((chunk * 16,), F32),
            pltpu.VMEM((zrows, 128), F32),
            pltpu.VMEM_SHARED((npad, 128), F32),
            pltpu.SemaphoreType.DMA,
            pltpu.SemaphoreType.DMA,
            pltpu.SemaphoreType.DMA,
            pltpu.SemaphoreType.DMA,
            pltpu.SemaphoreType.DMA,
            pltpu.SemaphoreType.DMA,
        ],
    )
    def k(score_hbm, smax_hbm, idx_hbm, out_hbm, ex_hbm,
          idx_all, ib0, ib1, sb0, sb1, sm0, sm1, vb, eb,
          zbuf, acc, sl0, sl1, sg0, sg1, ss, sw):
        cid = lax.axis_index("c")
        sid = lax.axis_index("s")
        ibs, sbs, sms = (ib0, ib1), (sb0, sb1), (sm0, sm1)
        sls, sgs = (sl0, sl1), (sg0, sg1)

        @pl.loop(0, zrows)
        def _(r):
            for cc in range(8):
                zbuf[r, pl.ds(cc * 16, 16)] = jnp.zeros((16,), F32)

        for kk in range(rows_per_sub // zrows):
            pltpu.sync_copy(
                zbuf, acc.at[pl.ds(sid * rows_per_sub + kk * zrows, zrows)])
        plsc.subcore_barrier()

        @pl.loop(0, chunk)
        def _(r):
            for cc in range(1, 8):
                vb[r, pl.ds(cc * 16, 16)] = jnp.zeros((16,), F32)

        base = cid * e2 + sid * per_w
        pltpu.sync_copy(idx_hbm.at[pl.ds(base, per_w)], idx_all)
        for o in range(0, chunk, 16):
            ib0[pl.ds(o, 16)] = idx_all[pl.ds(o, 16)]
        pltpu.async_copy(
            score_hbm.at[pl.ds(base * 16, chunk * 16)], sb0, sl0)
        pltpu.async_copy(smax_hbm.at[ib0], sm0, sg0)

        @pl.loop(0, n_chunks)
        def _(ci):
            off = base + ci * chunk
            for s in range(2):
                s2 = 1 - s

                @pl.when(ci % 2 == s)
                def _(s=s, s2=s2):
                    pltpu.make_async_copy(
                        score_hbm.at[pl.ds(off * 16, chunk * 16)],
                        sbs[s], sls[s]).wait()
                    pltpu.make_async_copy(
                        smax_hbm.at[pl.ds(0, chunk)], sms[s], sgs[s]).wait()

                    @pl.when(ci >= 1)
                    def _():
                        pltpu.make_async_copy(
                            smax_hbm.at[pl.ds(0, chunk)], vb, ss).wait()
                        pltpu.make_async_copy(
                            score_hbm.at[pl.ds(off * 16, chunk * 16)],
                            eb, sw).wait()

                    @pl.when(ci + 1 < n_chunks)
                    def _():
                        for o in range(0, chunk, 16):
                            ibs[s2][pl.ds(o, 16)] = (
                                idx_all[pl.ds((ci + 1) * chunk + o, 16)])
                        pltpu.async_copy(
                            score_hbm.at[
                                pl.ds((off + chunk) * 16, chunk * 16)],
                            sbs[s2], sls[s2])
                        pltpu.async_copy(
                            smax_hbm.at[ibs[s2]], sms[s2], sgs[s2])

                    @pl.loop(0, chunk)
                    def _(ei, s=s):
                        sv = sbs[s][pl.ds(ei * 16, 16)]
                        m = sms[s][ei, pl.ds(0, 16)]
                        ex = jnp.exp(sv - m)
                        vb[ei, pl.ds(0, 16)] = ex
                        eb[pl.ds(ei * 16, 16)] = ex

                    pltpu.async_copy(vb, acc.at[ibs[s]], ss, add=True)
                    pltpu.async_copy(
                        eb, ex_hbm.at[pl.ds(off * 16, chunk * 16)], sw)

        pltpu.make_async_copy(
            smax_hbm.at[pl.ds(0, chunk)], vb, ss).wait()
        pltpu.make_async_copy(
            score_hbm.at[pl.ds(base * 16, chunk * 16)], eb, sw).wait()

        plsc.subcore_barrier()
        for kk in range(rows_per_sub // zrows):
            rs = sid * rows_per_sub + kk * zrows
            pltpu.sync_copy(acc.at[pl.ds(rs, zrows)],
                            out_hbm.at[cid].at[pl.ds(rs, zrows)])

    den, ex_flat = k(score_flat, smax_pad, idx)
    return den[:, :n, :], ex_flat


def _tc_wv(ex_pk, vs3, be=3200):
    ep8 = ex_pk.shape[0]
    bp = be // 8

    def body(p_r, vs_r, o_r):
        p = p_r[...]
        a = lax.broadcasted_iota(jnp.int32, (128, 128), 0)
        j = lax.broadcasted_iota(jnp.int32, (128, 128), 1)
        parts = []
        for g in range(8):
            m = (a == g * 16 + j // 16).astype(F32)
            parts.append(_dot(p, m)[:, None, :])
        exb = jnp.concatenate(parts, axis=1)
        o_r[...] = vs_r[...] * exb

    return pl.pallas_call(
        body,
        grid=(ep8 // bp,),
        in_specs=[
            pl.BlockSpec((bp, 128), lambda i: (i, 0)),
            pl.BlockSpec((bp, 8, 128), lambda i: (i, 0, 0)),
        ],
        out_specs=pl.BlockSpec((bp, 8, 128), lambda i: (i, 0, 0)),
        out_shape=jax.ShapeDtypeStruct((ep8, 8, 128), F32),
    )(ex_pk, vs3)


def kernel(x, edge_index, dist_rbf, eig_pe, edge_attr, batch,
           W_phi1, b_phi1, W_phi2, b_phi2, W_edge, W_rho, b_rho,
           Wq, Wk, Wv, We, Wr1, br1, Wr2, br2):
    n = x.shape[0]
    e = edge_index.shape[1]
    src = edge_index[0]
    dst = edge_index[1]
    b_phi1 = b_phi1.reshape(1, -1)
    b_phi2 = b_phi2.reshape(1, -1)
    b_rho = b_rho.reshape(1, -1)
    br1 = br1.reshape(1, -1)
    br2 = br2.reshape(1, 1)
    batch3 = batch.reshape(n // 1000, 1, 1000)

    h = _tc_signnet_h(eig_pe, W_phi1, b_phi1, W_phi2, b_phi2)
    (hs,) = _sc_gather([h], [src], chunk=80)
    msg = _tc_msg(hs, edge_attr, W_edge)
    m_part = _sc_scatter_add(msg, dst, n)
    q, k, v = _tc_qkv(x, h, m_part, W_rho, b_rho, Wq, Wk, Wv)
    qd, ks, vs = _sc_gather([q, k, v], [dst, src, src], chunk=80)
    score = _tc_score(qd, ks, dist_rbf, We)
    score_flat = score.reshape(-1)
    smax_part = _sc_segmax(score_flat, dst, n)
    smax_pad = _tc_smax_merge(
        smax_part.reshape(NW, n // 16, 128)).reshape(n, 128)
    den_part, ex_flat = _sc_den_exp(score_flat, smax_pad, dst, n)
    wv = _tc_wv(ex_flat.reshape(e // 8, 128),
                vs.reshape(e // 8, 8, 128)).reshape(e, 128)
    agg_part = _sc_scatter_add(wv, dst, n)
    out = _tc_final(agg_part, den_part, batch3, Wr1, br1, Wr2, br2)
    return out


# speedup vs baseline: 16.6893x; 1.0065x over previous
import functools

import jax
import jax.numpy as jnp
from jax import lax
from jax.experimental import pallas as pl
from jax.experimental.pallas import tpu as pltpu
from jax.experimental.pallas import tpu_sc as plsc

F32 = jnp.float32
HI = jax.lax.Precision.HIGHEST

NC = 2
NS = 16
NW = NC * NS


def _dot(a, b):
    return jax.lax.dot(a, b, precision=HI, preferred_element_type=F32)


def _tc_signnet_h(pe, w1, b1, w2, b2):
    n = pe.shape[0]

    def body(pe_r, w1_r, b1_r, w2_r, b2_r, o_r):
        def phi(z):
            h1 = jnp.maximum(_dot(z, w1_r[...]) + b1_r[...], 0.0)
            return jnp.maximum(_dot(h1, w2_r[...]) + b2_r[...], 0.0)
        z = pe_r[...]
        o_r[...] = phi(z) + phi(-z)

    return pl.pallas_call(
        body,
        out_shape=jax.ShapeDtypeStruct((n, 128), F32),
    )(pe, w1, b1, w2, b2)


def _tc_er(edge_attr, w_edge, be=3200):
    e = edge_attr.shape[0]

    def body(ea_r, w_r, o_r):
        o_r[...] = jnp.maximum(_dot(ea_r[...], w_r[...]), 0.0)

    return pl.pallas_call(
        body,
        grid=(e // be,),
        in_specs=[
            pl.BlockSpec((be, 16), lambda i: (i, 0)),
            pl.BlockSpec((16, 128), lambda i: (0, 0)),
        ],
        out_specs=pl.BlockSpec((be, 128), lambda i: (i, 0)),
        out_shape=jax.ShapeDtypeStruct((e, 128), F32),
    )(edge_attr, w_edge)


def _tc_qkv(x, h, m_part, w_rho, b_rho, wq, wk, wv, bn=2000):
    n = x.shape[0]

    def body(x_r, h_r, mp_r, wr_r, br_r, wq_r, wk_r, wv_r, q_r, k_r, v_r):
        hm = h_r[...] + mp_r[0] + mp_r[1]
        pos = _dot(hm, wr_r[...]) + br_r[...]
        hn = x_r[...] + pos
        q_r[...] = _dot(hn, wq_r[...])
        k_r[...] = _dot(hn, wk_r[...])
        v_r[...] = _dot(hn, wv_r[...])

    spec = pl.BlockSpec((bn, 128), lambda i: (i, 0))
    wspec = pl.BlockSpec((128, 128), lambda i: (0, 0))
    return pl.pallas_call(
        body,
        grid=(n // bn,),
        in_specs=[
            spec,
            spec,
            pl.BlockSpec((2, bn, 128), lambda i: (0, i, 0)),
            wspec,
            pl.BlockSpec((1, 128), lambda i: (0, 0)),
            wspec, wspec, wspec,
        ],
        out_specs=[spec, spec, spec],
        out_shape=[jax.ShapeDtypeStruct((n, 128), F32)] * 3,
    )(x, h, m_part, w_rho, b_rho, wq, wk, wv)


def _tc_score(qd, ks, dist_rbf, we, be=3200):
    e = qd.shape[0]

    def body(qd_r, ks_r, rbf_r, we_r, o_r):
        eb = _dot(rbf_r[...], we_r[...])
        p = qd_r[...] * (ks_r[...] + eb)
        j = lax.broadcasted_iota(jnp.int32, (128, 16), 0)
        hcol = lax.broadcasted_iota(jnp.int32, (128, 16), 1)
        t = jnp.where((j // 16) == hcol, 0.25, 0.0).astype(F32)
        o_r[...] = _dot(p, t)

    return pl.pallas_call(
        body,
        grid=(e // be,),
        in_specs=[
            pl.BlockSpec((be, 128), lambda i: (i, 0)),
            pl.BlockSpec((be, 128), lambda i: (i, 0)),
            pl.BlockSpec((be, 16), lambda i: (i, 0)),
            pl.BlockSpec((16, 128), lambda i: (0, 0)),
        ],
        out_specs=pl.BlockSpec((be, 16), lambda i: (i, 0)),
        out_shape=jax.ShapeDtypeStruct((e, 16), F32),
    )(qd, ks, dist_rbf, we)


def _tc_smax_merge(smax_part, bm=625):
    npk = smax_part.shape[1]

    def body(sp_r, o_r):
        pk = jnp.max(sp_r[...], axis=0)
        z = jnp.zeros((pk.shape[0], 120), F32)
        parts = [
            jnp.concatenate([pk[:, g * 8:(g + 1) * 8], z], axis=1)[:, None, :]
            for g in range(16)
        ]
        o_r[...] = jnp.concatenate(parts, axis=1)

    return pl.pallas_call(
        body,
        grid=(npk // bm,),
        in_specs=[pl.BlockSpec((NW, bm, 128), lambda i: (0, i, 0))],
        out_specs=pl.BlockSpec((bm, 16, 128), lambda i: (i, 0, 0)),
        out_shape=jax.ShapeDtypeStruct((npk, 16, 128), F32),
    )(smax_part)


def _tc_final(agg_part, den_part, batch3, wr1, br1, wr2, br2, bn=1000):
    n = agg_part.shape[1]
    nblk = n // bn

    def body(ap_r, dp_r, b_r, wr1_r, br1_r, wr2_r, br2_r, o_r, gsum, cnt):
        i = pl.program_id(0)

        @pl.when(i == 0)
        def _():
            gsum[...] = jnp.zeros_like(gsum)
            cnt[...] = jnp.zeros_like(cnt)

        a = ap_r[0] + ap_r[1]
        dsum = dp_r[0, :, :16] + dp_r[1, :, :16]
        hrow = lax.broadcasted_iota(jnp.int32, (16, 128), 0)
        j = lax.broadcasted_iota(jnp.int32, (16, 128), 1)
        r = jnp.where((j // 16) == hrow, 1.0, 0.0).astype(F32)
        den = _dot(dsum, r) + 1e-9
        node = a / den

        ids = b_r[0]
        g = lax.broadcasted_iota(jnp.int32, (128, bn), 0)
        oh = (jnp.broadcast_to(ids, (128, bn)) == g).astype(F32)
        gsum[...] += _dot(oh, node)
        cnt[...] += jnp.sum(oh, axis=1, keepdims=True)

        @pl.when(i == nblk - 1)
        def _():
            graph = gsum[...] / jnp.maximum(cnt[...], 1.0)
            og = jnp.maximum(_dot(graph, wr1_r[...]) + br1_r[...], 0.0)
            o_r[...] = _dot(og, wr2_r[...]) + br2_r[...]

    return pl.pallas_call(
        body,
        grid=(nblk,),
        in_specs=[
            pl.BlockSpec((2, bn, 128), lambda i: (0, i, 0)),
            pl.BlockSpec((2, bn, 128), lambda i: (0, i, 0)),
            pl.BlockSpec((1, 1, bn), lambda i: (i, 0, 0)),
            pl.BlockSpec((128, 128), lambda i: (0, 0)),
            pl.BlockSpec((1, 128), lambda i: (0, 0)),
            pl.BlockSpec((128, 1), lambda i: (0, 0)),
            pl.BlockSpec((1, 1), lambda i: (0, 0)),
        ],
        out_specs=pl.BlockSpec((128, 1), lambda i: (0, 0)),
        out_shape=jax.ShapeDtypeStruct((128, 1), F32),
        scratch_shapes=[
            pltpu.VMEM((128, 128), F32),
            pltpu.VMEM((128, 1), F32),
        ],
    )(agg_part, den_part, batch3, wr1, br1, wr2, br2)


_MESH = plsc.VectorSubcoreMesh(core_axis_name="c", subcore_axis_name="s")


def _sc_gather(tables, idxs, chunk=80):
    e = idxs[0].shape[0]
    per_w = e // NW
    n_chunks = per_w // chunk
    t_count = len(tables)

    scratch = []
    for _ in range(t_count):
        scratch.append(pltpu.VMEM((per_w,), jnp.int32))
        scratch.append(pltpu.VMEM((chunk, 128), F32))
        scratch.append(pltpu.VMEM((chunk, 128), F32))
        scratch.append(pltpu.SemaphoreType.DMA)
        scratch.append(pltpu.SemaphoreType.DMA)
        scratch.append(pltpu.VMEM((chunk,), jnp.int32))
        scratch.append(pltpu.VMEM((chunk,), jnp.int32))

    @functools.partial(
        pl.kernel,
        out_type=[jax.ShapeDtypeStruct((e, 128), F32) for _ in range(t_count)],
        mesh=_MESH,
        scratch_types=scratch,
    )
    def k(*refs):
        tabs = refs[:t_count]
        idx_hbm = refs[t_count:2 * t_count]
        outs = refs[2 * t_count:3 * t_count]
        sc = refs[3 * t_count:]
        cid = lax.axis_index("c")
        sid = lax.axis_index("s")
        base = (cid * NS + sid) * per_w

        for t in range(t_count):
            pltpu.sync_copy(idx_hbm[t].at[pl.ds(base, per_w)], sc[7 * t])

        @pl.loop(0, n_chunks)
        def _(ci):
            off = base + ci * chunk
            for t in range(t_count):
                idx_all = sc[7 * t]
                for s in range(2):
                    buf = sc[7 * t + 1 + s]
                    sem = sc[7 * t + 3 + s]
                    ib = sc[7 * t + 5 + s]

                    @pl.when(ci % 2 == s)
                    def _(buf=buf, sem=sem, ib=ib, t=t):
                        @pl.when(ci >= 2)
                        def _():
                            pltpu.make_async_copy(
                                outs[t].at[pl.ds(off, chunk)], buf, sem
                            ).wait()
                        for o in range(0, chunk, 16):
                            ib[pl.ds(o, 16)] = (
                                idx_all[pl.ds(ci * chunk + o, 16)])
                        pltpu.async_copy(tabs[t].at[ib], buf, sem).wait()
                        pltpu.async_copy(
                            buf, outs[t].at[pl.ds(off, chunk)], sem)

        for t in range(t_count):
            for s in range(2):
                buf = sc[7 * t + 1 + s]
                sem = sc[7 * t + 3 + s]
                pltpu.make_async_copy(
                    outs[t].at[pl.ds(base, chunk)], buf, sem).wait()

    res = k(*tables, *idxs)
    if not isinstance(res, (list, tuple)):
        res = [res]
    return list(res)


def _sc_msg_fused(h, er, src, dst, n, chunk=80):
    e = src.shape[0]
    e2 = e // NC
    per_w = e2 // NS
    n_chunks = per_w // chunk
    npad = 10240
    rows_per_sub = npad // NS
    zrows = 32

    @functools.partial(
        pl.kernel,
        out_type=jax.ShapeDtypeStruct((NC, npad, 128), F32),
        mesh=_MESH,
        scratch_types=[
            pltpu.VMEM((chunk,), jnp.int32),
            pltpu.VMEM((chunk,), jnp.int32),
            pltpu.VMEM((chunk,), jnp.int32),
            pltpu.VMEM((chunk,), jnp.int32),
            pltpu.VMEM((chunk, 128), F32),
            pltpu.VMEM((chunk, 128), F32),
            pltpu.VMEM((chunk, 128), F32),
            pltpu.VMEM((chunk, 128), F32),
            pltpu.VMEM((zrows, 128), F32),
            pltpu.VMEM_SHARED((npad, 128), F32),
            pltpu.SemaphoreType.DMA,
            pltpu.SemaphoreType.DMA,
            pltpu.SemaphoreType.DMA,
            pltpu.SemaphoreType.DMA,
            pltpu.SemaphoreType.DMA,
            pltpu.SemaphoreType.DMA,
            pltpu.SemaphoreType.DMA,
            pltpu.SemaphoreType.DMA,
            pltpu.SemaphoreType.DMA,
        ],
    )
    def k(h_hbm, er_hbm, src_hbm, dst_hbm, out_hbm,
          sib0, sib1, dib0, dib1, gb0, gb1, eb0, eb1, zbuf, acc,
          ssi0, ssi1, sdi0, sdi1, sg0, sg1, sel0, sel1, ss):
        cid = lax.axis_index("c")
        sid = lax.axis_index("s")
        sibs, dibs = (sib0, sib1), (dib0, dib1)
        gbs, ebs = (gb0, gb1), (eb0, eb1)
        ssis, sdis = (ssi0, ssi1), (sdi0, sdi1)
        sgs, sels = (sg0, sg1), (sel0, sel1)

        @pl.loop(0, zrows)
        def _(r):
            for cc in range(8):
                zbuf[r, pl.ds(cc * 16, 16)] = jnp.zeros((16,), F32)

        for kk in range(rows_per_sub // zrows):
            pltpu.sync_copy(
                zbuf, acc.at[pl.ds(sid * rows_per_sub + kk * zrows, zrows)])
        plsc.subcore_barrier()

        base = cid * e2 + sid * per_w
        pltpu.sync_copy(src_hbm.at[pl.ds(base, chunk)], sib0)
        pltpu.sync_copy(dst_hbm.at[pl.ds(base, chunk)], dib0)
        pltpu.async_copy(h_hbm.at[sib0], gb0, sg0)
        pltpu.async_copy(er_hbm.at[pl.ds(base, chunk)], eb0, sel0)

        @pl.loop(0, n_chunks)
        def _(ci):
            off = base + ci * chunk
            for s in range(2):
                s2 = 1 - s

                @pl.when(ci % 2 == s)
                def _(s=s, s2=s2):
                    @pl.when(ci >= 1)
                    def _():
                        pltpu.make_async_copy(
                            h_hbm.at[pl.ds(0, chunk)], gbs[s2], ss).wait()

                    @pl.when(ci + 1 < n_chunks)
                    def _():
                        pltpu.async_copy(
                            src_hbm.at[pl.ds(off + chunk, chunk)],
                            sibs[s2], ssis[s2])
                        pltpu.async_copy(
                            dst_hbm.at[pl.ds(off + chunk, chunk)],
                            dibs[s2], sdis[s2])
                        pltpu.async_copy(
                            er_hbm.at[pl.ds(off + chunk, chunk)],
                            ebs[s2], sels[s2])

                    pltpu.make_async_copy(
                        h_hbm.at[pl.ds(0, chunk)], gbs[s], sgs[s]).wait()
                    pltpu.make_async_copy(
                        er_hbm.at[pl.ds(0, chunk)], ebs[s], sels[s]).wait()

                    @pl.loop(0, chunk)
                    def _(ei, s=s):
                        for cc in range(8):
                            g = gbs[s][ei, pl.ds(cc * 16, 16)]
                            ee = ebs[s][ei, pl.ds(cc * 16, 16)]
                            gbs[s][ei, pl.ds(cc * 16, 16)] = (
                                jnp.maximum(g + ee, 0.0))

                    pltpu.async_copy(gbs[s], acc.at[dibs[s]], ss, add=True)

                    @pl.when(ci + 1 < n_chunks)
                    def _():
                        pltpu.make_async_copy(
                            src_hbm.at[pl.ds(0, chunk)],
                            sibs[s2], ssis[s2]).wait()
                        pltpu.make_async_copy(
                            dst_hbm.at[pl.ds(0, chunk)],
                            dibs[s2], sdis[s2]).wait()
                        pltpu.async_copy(h_hbm.at[sibs[s2]], gbs[s2], sgs[s2])

        ls = (n_chunks - 1) % 2
        pltpu.make_async_copy(
            h_hbm.at[pl.ds(0, chunk)], gbs[ls], ss).wait()

        plsc.subcore_barrier()
        for kk in range(rows_per_sub // zrows):
            rs = sid * rows_per_sub + kk * zrows
            pltpu.sync_copy(acc.at[pl.ds(rs, zrows)],
                            out_hbm.at[cid].at[pl.ds(rs, zrows)])

    return k(h, er, src, dst)[:, :n, :]


def _sc_scatter_add(values, idx, n, chunk=80):
    e, width = values.shape
    e2 = e // NC
    per_w = e2 // NS
    n_chunks = per_w // chunk
    npad = 10240
    rows_per_sub = npad // NS
    zrows = rows_per_sub // 5

    @functools.partial(
        pl.kernel,
        out_type=jax.ShapeDtypeStruct((NC, npad, width), F32),
        mesh=_MESH,
        scratch_types=[
            pltpu.VMEM((per_w,), jnp.int32),
            pltpu.VMEM((chunk,), jnp.int32),
            pltpu.VMEM((chunk,), jnp.int32),
            pltpu.VMEM((chunk, width), F32),
            pltpu.VMEM((chunk, width), F32),
            pltpu.VMEM((zrows, width), F32),
            pltpu.VMEM_SHARED((npad, width), F32),
            pltpu.SemaphoreType.DMA,
            pltpu.SemaphoreType.DMA,
            pltpu.SemaphoreType.DMA,
            pltpu.SemaphoreType.DMA,
        ],
    )
    def k(val_hbm, idx_hbm, out_hbm, idx_all, ib0, ib1, vb0, vb1, zbuf,
          acc, sl0, sl1, ss0, ss1):
        cid = lax.axis_index("c")
        sid = lax.axis_index("s")
        ibs, vbs, sls, sss = (ib0, ib1), (vb0, vb1), (sl0, sl1), (ss0, ss1)

        @pl.loop(0, zrows)
        def _(r):
            for cc in range(width // 16):
                zbuf[r, pl.ds(cc * 16, 16)] = jnp.zeros((16,), F32)

        for kk in range(5):
            pltpu.sync_copy(
                zbuf, acc.at[pl.ds(sid * rows_per_sub + kk * zrows, zrows)])
        plsc.subcore_barrier()

        base = cid * e2 + sid * per_w
        pltpu.sync_copy(idx_hbm.at[pl.ds(base, per_w)], idx_all)
        pltpu.async_copy(val_hbm.at[pl.ds(base, chunk)], vb0, sl0)

        @pl.loop(0, n_chunks)
        def _(ci):
            off = base + ci * chunk
            for s in range(2):
                s2 = 1 - s

                @pl.when(ci % 2 == s)
                def _(s=s, s2=s2):
                    @pl.when(ci >= 1)
                    def _():
                        pltpu.make_async_copy(
                            val_hbm.at[pl.ds(off, chunk)], vbs[s2], sss[s2]
                        ).wait()

                    @pl.when(ci + 1 < n_chunks)
                    def _():
                        pltpu.async_copy(
                            val_hbm.at[pl.ds(off + chunk, chunk)],
                            vbs[s2], sls[s2])

                    pltpu.make_async_copy(
                        val_hbm.at[pl.ds(off, chunk)], vbs[s], sls[s]
                    ).wait()
                    for o in range(0, chunk, 16):
                        ibs[s][pl.ds(o, 16)] = (
                            idx_all[pl.ds(ci * chunk + o, 16)])
                    pltpu.async_copy(
                        vbs[s], acc.at[ibs[s]], sss[s], add=True)

        ls = (n_chunks - 1) % 2
        pltpu.make_async_copy(
            val_hbm.at[pl.ds(base, chunk)], vbs[ls], sss[ls]).wait()

        plsc.subcore_barrier()
        for kk in range(5):
            rs = sid * rows_per_sub + kk * zrows
            pltpu.sync_copy(acc.at[pl.ds(rs, zrows)],
                            out_hbm.at[cid].at[pl.ds(rs, zrows)])

    return k(values, idx)[:, :n, :]


def _sc_segmax(score_flat, idx, n, chunk=1000):
    e = idx.shape[0]
    per_w = e // NW
    n_chunks = per_w // chunk
    tsz = n * 8 + 16

    @functools.partial(
        pl.kernel,
        out_type=jax.ShapeDtypeStruct((NW * n * 8,), F32),
        mesh=_MESH,
        scratch_types=[
            pltpu.VMEM((chunk + 16,), jnp.int32),
            pltpu.VMEM((chunk * 16,), F32),
            pltpu.VMEM((tsz,), F32),
        ],
        compiler_params=pltpu.CompilerParams(needs_layout_passes=False),
    )
    def k(score_hbm, idx_hbm, out_hbm, idxbuf, scorebuf, table):
        cid = lax.axis_index("c")
        sid = lax.axis_index("s")
        wid = cid * NS + sid
        base = wid * per_w

        @pl.loop(0, tsz, step=16)
        def _(i):
            table[pl.ds(i, 16)] = jnp.full((16,), -3.0e38, F32)

        lanes = lax.iota(jnp.int32, 16)
        mask = lanes < 8

        @pl.loop(0, n_chunks)
        def _(ci):
            off = base + ci * chunk
            pltpu.sync_copy(score_hbm.at[pl.ds(off * 16, chunk * 16)],
                            scorebuf)
            pltpu.sync_copy(idx_hbm.at[pl.ds(off, chunk)],
                            idxbuf.at[pl.ds(0, chunk)])

            @pl.loop(0, chunk)
            def _(ei):
                d = idxbuf[pl.ds(ei, 16)][0]
                sv = scorebuf[pl.ds(ei * 16, 16)]
                iv = d * 8 + lanes
                cur = plsc.load_gather(table, [iv], mask=mask)
                plsc.store_scatter(table, [iv],
                                   jnp.maximum(cur, sv), mask=mask)

        pltpu.sync_copy(table.at[pl.ds(0, n * 8)],
                        out_hbm.at[pl.ds(wid * n * 8, n * 8)])

    return k(score_flat, idx)


def _sc_den_exp(score_flat, smax_pad, idx, n, chunk=80):
    e = idx.shape[0]
    e2 = e // NC
    per_w = e2 // NS
    n_chunks = per_w // chunk
    npad = 10240
    rows_per_sub = npad // NS
    zrows = 32

    @functools.partial(
        pl.kernel,
        out_type=[
            jax.ShapeDtypeStruct((NC, npad, 128), F32),
            jax.ShapeDtypeStruct((e * 16,), F32),
        ],
        mesh=_MESH,
        scratch_types=[
            pltpu.VMEM((per_w,), jnp.int32),
            pltpu.VMEM((chunk,), jnp.int32),
            pltpu.VMEM((chunk,), jnp.int32),
            pltpu.VMEM((chunk * 16,), F32),
            pltpu.VMEM((chunk * 16,), F32),
            pltpu.VMEM((chunk, 128), F32),
            pltpu.VMEM((chunk, 128), F32),
            pltpu.VMEM((chunk, 128), F32),
            pltpu.VMEM((chunk * 16,), F32),
            pltpu.VMEM((zrows, 128), F32),
            pltpu.VMEM_SHARED((npad, 128), F32),
            pltpu.SemaphoreType.DMA,
            pltpu.SemaphoreType.DMA,
            pltpu.SemaphoreType.DMA,
            pltpu.SemaphoreType.DMA,
            pltpu.SemaphoreType.DMA,
            pltpu.SemaphoreType.DMA,
        ],
    )
    def k(score_hbm, smax_hbm, idx_hbm, out_hbm, ex_hbm,
          idx_all, ib0, ib1, sb0, sb1, sm0, sm1, vb, eb,
          zbuf, acc, sl0, sl1, sg0, sg1, ss, sw):
        cid = lax.axis_index("c")
        sid = lax.axis_index("s")
        ibs, sbs, sms = (ib0, ib1), (sb0, sb1), (sm0, sm1)
        sls, sgs = (sl0, sl1), (sg0, sg1)

        @pl.loop(0, zrows)
        def _(r):
            for cc in range(8):
                zbuf[r, pl.ds(cc * 16, 16)] = jnp.zeros((16,), F32)

        for kk in range(rows_per_sub // zrows):
            pltpu.sync_copy(
                zbuf, acc.at[pl.ds(sid * rows_per_sub + kk * zrows, zrows)])
        plsc.subcore_barrier()

        @pl.loop(0, chunk)
        def _(r):
            for cc in range(1, 8):
                vb[r, pl.ds(cc * 16, 16)] = jnp.zeros((16,), F32)

        base = cid * e2 + sid * per_w
        pltpu.sync_copy(idx_hbm.at[pl.ds(base, per_w)], idx_all)
        for o in range(0, chunk, 16):
            ib0[pl.ds(o, 16)] = idx_all[pl.ds(o, 16)]
        pltpu.async_copy(
            score_hbm.at[pl.ds(base * 16, chunk * 16)], sb0, sl0)
        pltpu.async_copy(smax_hbm.at[ib0], sm0, sg0)

        @pl.loop(0, n_chunks)
        def _(ci):
            off = base + ci * chunk
            for s in range(2):
                s2 = 1 - s

                @pl.when(ci % 2 == s)
                def _(s=s, s2=s2):
                    pltpu.make_async_copy(
                        score_hbm.at[pl.ds(off * 16, chunk * 16)],
                        sbs[s], sls[s]).wait()
                    pltpu.make_async_copy(
                        smax_hbm.at[pl.ds(0, chunk)], sms[s], sgs[s]).wait()

                    @pl.when(ci >= 1)
                    def _():
                        pltpu.make_async_copy(
                            smax_hbm.at[pl.ds(0, chunk)], vb, ss).wait()
                        pltpu.make_async_copy(
                            score_hbm.at[pl.ds(off * 16, chunk * 16)],
                            eb, sw).wait()

                    @pl.when(ci + 1 < n_chunks)
                    def _():
                        for o in range(0, chunk, 16):
                            ibs[s2][pl.ds(o, 16)] = (
                                idx_all[pl.ds((ci + 1) * chunk + o, 16)])
                        pltpu.async_copy(
                            score_hbm.at[
                                pl.ds((off + chunk) * 16, chunk * 16)],
                            sbs[s2], sls[s2])
                        pltpu.async_copy(
                            smax_hbm.at[ibs[s2]], sms[s2], sgs[s2])

                    @pl.loop(0, chunk)
                    def _(ei, s=s):
                        sv = sbs[s][pl.ds(ei * 16, 16)]
                        m = sms[s][ei, pl.ds(0, 16)]
                        ex = jnp.exp(sv - m)
                        vb[ei, pl.ds(0, 16)] = ex
                        eb[pl.ds(ei * 16, 16)] = ex

                    pltpu.async_copy(vb, acc.at[ibs[s]], ss, add=True)
                    pltpu.async_copy(
                        eb, ex_hbm.at[pl.ds(off * 16, chunk * 16)], sw)

        pltpu.make_async_copy(
            smax_hbm.at[pl.ds(0, chunk)], vb, ss).wait()
        pltpu.make_async_copy(
            score_hbm.at[pl.ds(base * 16, chunk * 16)], eb, sw).wait()

        plsc.subcore_barrier()
        for kk in range(rows_per_sub // zrows):
            rs = sid * rows_per_sub + kk * zrows
            pltpu.sync_copy(acc.at[pl.ds(rs, zrows)],
                            out_hbm.at[cid].at[pl.ds(rs, zrows)])

    den, ex_flat = k(score_flat, smax_pad, idx)
    return den[:, :n, :], ex_flat


def _tc_wv(ex_pk, vs3, be=3200):
    ep8 = ex_pk.shape[0]
    bp = be // 8

    def body(p_r, vs_r, o_r):
        p = p_r[...]
        a = lax.broadcasted_iota(jnp.int32, (128, 128), 0)
        j = lax.broadcasted_iota(jnp.int32, (128, 128), 1)
        parts = []
        for g in range(8):
            m = (a == g * 16 + j // 16).astype(F32)
            parts.append(_dot(p, m)[:, None, :])
        exb = jnp.concatenate(parts, axis=1)
        o_r[...] = vs_r[...] * exb

    return pl.pallas_call(
        body,
        grid=(ep8 // bp,),
        in_specs=[
            pl.BlockSpec((bp, 128), lambda i: (i, 0)),
            pl.BlockSpec((bp, 8, 128), lambda i: (i, 0, 0)),
        ],
        out_specs=pl.BlockSpec((bp, 8, 128), lambda i: (i, 0, 0)),
        out_shape=jax.ShapeDtypeStruct((ep8, 8, 128), F32),
    )(ex_pk, vs3)


def kernel(x, edge_index, dist_rbf, eig_pe, edge_attr, batch,
           W_phi1, b_phi1, W_phi2, b_phi2, W_edge, W_rho, b_rho,
           Wq, Wk, Wv, We, Wr1, br1, Wr2, br2):
    n = x.shape[0]
    e = edge_index.shape[1]
    src = edge_index[0]
    dst = edge_index[1]
    b_phi1 = b_phi1.reshape(1, -1)
    b_phi2 = b_phi2.reshape(1, -1)
    b_rho = b_rho.reshape(1, -1)
    br1 = br1.reshape(1, -1)
    br2 = br2.reshape(1, 1)
    batch3 = batch.reshape(n // 1000, 1, 1000)

    h = _tc_signnet_h(eig_pe, W_phi1, b_phi1, W_phi2, b_phi2)
    er = _tc_er(edge_attr, W_edge)
    m_part = _sc_msg_fused(h, er, src, dst, n)
    q, k, v = _tc_qkv(x, h, m_part, W_rho, b_rho, Wq, Wk, Wv)
    qd, ks, vs = _sc_gather([q, k, v], [dst, src, src], chunk=80)
    score = _tc_score(qd, ks, dist_rbf, We)
    score_flat = score.reshape(-1)
    smax_part = _sc_segmax(score_flat, dst, n)
    smax_pad = _tc_smax_merge(
        smax_part.reshape(NW, n // 16, 128)).reshape(n, 128)
    den_part, ex_flat = _sc_den_exp(score_flat, smax_pad, dst, n)
    wv = _tc_wv(ex_flat.reshape(e // 8, 128),
                vs.reshape(e // 8, 8, 128)).reshape(e, 128)
    agg_part = _sc_scatter_add(wv, dst, n)
    out = _tc_final(agg_part, den_part, batch3, Wr1, br1, Wr2, br2)
    return out


# speedup vs baseline: 16.8623x; 1.0104x over previous
import functools

import jax
import jax.numpy as jnp
from jax import lax
from jax.experimental import pallas as pl
from jax.experimental.pallas import tpu as pltpu
from jax.experimental.pallas import tpu_sc as plsc

F32 = jnp.float32
HI = jax.lax.Precision.HIGHEST

NC = 2
NS = 16
NW = NC * NS


def _dot(a, b):
    return jax.lax.dot(a, b, precision=HI, preferred_element_type=F32)


def _tc_signnet_h(pe, w1, b1, w2, b2):
    n = pe.shape[0]

    def body(pe_r, w1_r, b1_r, w2_r, b2_r, o_r):
        def phi(z):
            h1 = jnp.maximum(_dot(z, w1_r[...]) + b1_r[...], 0.0)
            return jnp.maximum(_dot(h1, w2_r[...]) + b2_r[...], 0.0)
        z = pe_r[...]
        o_r[...] = phi(z) + phi(-z)

    return pl.pallas_call(
        body,
        out_shape=jax.ShapeDtypeStruct((n, 128), F32),
    )(pe, w1, b1, w2, b2)


def _tc_er(edge_attr, w_edge, be=3200):
    e = edge_attr.shape[0]

    def body(ea_r, w_r, o_r):
        o_r[...] = jnp.maximum(_dot(ea_r[...], w_r[...]), 0.0)

    return pl.pallas_call(
        body,
        grid=(e // be,),
        in_specs=[
            pl.BlockSpec((be, 16), lambda i: (i, 0)),
            pl.BlockSpec((16, 128), lambda i: (0, 0)),
        ],
        out_specs=pl.BlockSpec((be, 128), lambda i: (i, 0)),
        out_shape=jax.ShapeDtypeStruct((e, 128), F32),
    )(edge_attr, w_edge)


def _tc_qkv(x, h, m_part, w_rho, b_rho, wq, wk, wv, bn=2000):
    n = x.shape[0]

    def body(x_r, h_r, mp_r, wr_r, br_r, wq_r, wk_r, wv_r, q_r, k_r, v_r):
        hm = h_r[...] + mp_r[0] + mp_r[1]
        pos = _dot(hm, wr_r[...]) + br_r[...]
        hn = x_r[...] + pos
        q_r[...] = _dot(hn, wq_r[...])
        k_r[...] = _dot(hn, wk_r[...])
        v_r[...] = _dot(hn, wv_r[...])

    spec = pl.BlockSpec((bn, 128), lambda i: (i, 0))
    wspec = pl.BlockSpec((128, 128), lambda i: (0, 0))
    return pl.pallas_call(
        body,
        grid=(n // bn,),
        in_specs=[
            spec,
            spec,
            pl.BlockSpec((2, bn, 128), lambda i: (0, i, 0)),
            wspec,
            pl.BlockSpec((1, 128), lambda i: (0, 0)),
            wspec, wspec, wspec,
        ],
        out_specs=[spec, spec, spec],
        out_shape=[jax.ShapeDtypeStruct((n, 128), F32)] * 3,
    )(x, h, m_part, w_rho, b_rho, wq, wk, wv)


def _tc_score(qd, ks, dist_rbf, we, be=3200):
    e = qd.shape[0]

    def body(qd_r, ks_r, rbf_r, we_r, o_r):
        eb = _dot(rbf_r[...], we_r[...])
        p = qd_r[...] * (ks_r[...] + eb)
        j = lax.broadcasted_iota(jnp.int32, (128, 16), 0)
        hcol = lax.broadcasted_iota(jnp.int32, (128, 16), 1)
        t = jnp.where((j // 16) == hcol, 0.25, 0.0).astype(F32)
        o_r[...] = _dot(p, t)

    return pl.pallas_call(
        body,
        grid=(e // be,),
        in_specs=[
            pl.BlockSpec((be, 128), lambda i: (i, 0)),
            pl.BlockSpec((be, 128), lambda i: (i, 0)),
            pl.BlockSpec((be, 16), lambda i: (i, 0)),
            pl.BlockSpec((16, 128), lambda i: (0, 0)),
        ],
        out_specs=pl.BlockSpec((be, 16), lambda i: (i, 0)),
        out_shape=jax.ShapeDtypeStruct((e, 16), F32),
    )(qd, ks, dist_rbf, we)


def _tc_smax_merge(smax_part, bm=625):
    npk = smax_part.shape[1]

    def body(sp_r, o_r):
        pk = jnp.max(sp_r[...], axis=0)
        z = jnp.zeros((pk.shape[0], 120), F32)
        parts = [
            jnp.concatenate([pk[:, g * 8:(g + 1) * 8], z], axis=1)[:, None, :]
            for g in range(16)
        ]
        o_r[...] = jnp.concatenate(parts, axis=1)

    return pl.pallas_call(
        body,
        grid=(npk // bm,),
        in_specs=[pl.BlockSpec((NW, bm, 128), lambda i: (0, i, 0))],
        out_specs=pl.BlockSpec((bm, 16, 128), lambda i: (i, 0, 0)),
        out_shape=jax.ShapeDtypeStruct((npk, 16, 128), F32),
    )(smax_part)


def _tc_final(agg_part, den_part, batch3, wr1, br1, wr2, br2, bn=1000):
    n = agg_part.shape[1]
    nblk = n // bn

    def body(ap_r, dp_r, b_r, wr1_r, br1_r, wr2_r, br2_r, o_r, gsum, cnt):
        i = pl.program_id(0)

        @pl.when(i == 0)
        def _():
            gsum[...] = jnp.zeros_like(gsum)
            cnt[...] = jnp.zeros_like(cnt)

        a = ap_r[0] + ap_r[1]
        dsum = dp_r[0, :, :16] + dp_r[1, :, :16]
        hrow = lax.broadcasted_iota(jnp.int32, (16, 128), 0)
        j = lax.broadcasted_iota(jnp.int32, (16, 128), 1)
        r = jnp.where((j // 16) == hrow, 1.0, 0.0).astype(F32)
        den = _dot(dsum, r) + 1e-9
        node = a / den

        ids = b_r[0]
        g = lax.broadcasted_iota(jnp.int32, (128, bn), 0)
        oh = (jnp.broadcast_to(ids, (128, bn)) == g).astype(F32)
        gsum[...] += _dot(oh, node)
        cnt[...] += jnp.sum(oh, axis=1, keepdims=True)

        @pl.when(i == nblk - 1)
        def _():
            graph = gsum[...] / jnp.maximum(cnt[...], 1.0)
            og = jnp.maximum(_dot(graph, wr1_r[...]) + br1_r[...], 0.0)
            o_r[...] = _dot(og, wr2_r[...]) + br2_r[...]

    return pl.pallas_call(
        body,
        grid=(nblk,),
        in_specs=[
            pl.BlockSpec((2, bn, 128), lambda i: (0, i, 0)),
            pl.BlockSpec((2, bn, 128), lambda i: (0, i, 0)),
            pl.BlockSpec((1, 1, bn), lambda i: (i, 0, 0)),
            pl.BlockSpec((128, 128), lambda i: (0, 0)),
            pl.BlockSpec((1, 128), lambda i: (0, 0)),
            pl.BlockSpec((128, 1), lambda i: (0, 0)),
            pl.BlockSpec((1, 1), lambda i: (0, 0)),
        ],
        out_specs=pl.BlockSpec((128, 1), lambda i: (0, 0)),
        out_shape=jax.ShapeDtypeStruct((128, 1), F32),
        scratch_shapes=[
            pltpu.VMEM((128, 128), F32),
            pltpu.VMEM((128, 1), F32),
        ],
    )(agg_part, den_part, batch3, wr1, br1, wr2, br2)


_MESH = plsc.VectorSubcoreMesh(core_axis_name="c", subcore_axis_name="s")


def _sc_gather(tables, idxs, chunk=80):
    e = idxs[0].shape[0]
    per_w = e // NW
    n_chunks = per_w // chunk
    t_count = len(tables)

    scratch = []
    for _ in range(t_count):
        scratch.append(pltpu.VMEM((per_w,), jnp.int32))
        scratch.append(pltpu.VMEM((chunk, 128), F32))
        scratch.append(pltpu.VMEM((chunk, 128), F32))
        scratch.append(pltpu.SemaphoreType.DMA)
        scratch.append(pltpu.SemaphoreType.DMA)
        scratch.append(pltpu.VMEM((chunk,), jnp.int32))
        scratch.append(pltpu.VMEM((chunk,), jnp.int32))

    @functools.partial(
        pl.kernel,
        out_type=[jax.ShapeDtypeStruct((e, 128), F32) for _ in range(t_count)],
        mesh=_MESH,
        scratch_types=scratch,
    )
    def k(*refs):
        tabs = refs[:t_count]
        idx_hbm = refs[t_count:2 * t_count]
        outs = refs[2 * t_count:3 * t_count]
        sc = refs[3 * t_count:]
        cid = lax.axis_index("c")
        sid = lax.axis_index("s")
        base = (cid * NS + sid) * per_w

        for t in range(t_count):
            pltpu.sync_copy(idx_hbm[t].at[pl.ds(base, per_w)], sc[7 * t])

        @pl.loop(0, n_chunks)
        def _(ci):
            off = base + ci * chunk
            for t in range(t_count):
                idx_all = sc[7 * t]
                for s in range(2):
                    buf = sc[7 * t + 1 + s]
                    sem = sc[7 * t + 3 + s]
                    ib = sc[7 * t + 5 + s]

                    @pl.when(ci % 2 == s)
                    def _(buf=buf, sem=sem, ib=ib, t=t):
                        @pl.when(ci >= 2)
                        def _():
                            pltpu.make_async_copy(
                                outs[t].at[pl.ds(off, chunk)], buf, sem
                            ).wait()
                        for o in range(0, chunk, 16):
                            ib[pl.ds(o, 16)] = (
                                idx_all[pl.ds(ci * chunk + o, 16)])
                        pltpu.async_copy(tabs[t].at[ib], buf, sem).wait()
                        pltpu.async_copy(
                            buf, outs[t].at[pl.ds(off, chunk)], sem)

        for t in range(t_count):
            for s in range(2):
                buf = sc[7 * t + 1 + s]
                sem = sc[7 * t + 3 + s]
                pltpu.make_async_copy(
                    outs[t].at[pl.ds(base, chunk)], buf, sem).wait()

    res = k(*tables, *idxs)
    if not isinstance(res, (list, tuple)):
        res = [res]
    return list(res)


def _sc_msg_fused(h, er, src, dst, n, chunk=80):
    e = src.shape[0]
    e2 = e // NC
    per_w = e2 // NS
    n_chunks = per_w // chunk
    npad = 10240
    rows_per_sub = npad // NS
    zrows = 32

    @functools.partial(
        pl.kernel,
        out_type=jax.ShapeDtypeStruct((NC, npad, 128), F32),
        mesh=_MESH,
        scratch_types=[
            pltpu.VMEM((chunk,), jnp.int32),
            pltpu.VMEM((chunk,), jnp.int32),
            pltpu.VMEM((chunk,), jnp.int32),
            pltpu.VMEM((chunk,), jnp.int32),
            pltpu.VMEM((chunk, 128), F32),
            pltpu.VMEM((chunk, 128), F32),
            pltpu.VMEM((chunk, 128), F32),
            pltpu.VMEM((chunk, 128), F32),
            pltpu.VMEM((zrows, 128), F32),
            pltpu.VMEM_SHARED((npad, 128), F32),
            pltpu.SemaphoreType.DMA,
            pltpu.SemaphoreType.DMA,
            pltpu.SemaphoreType.DMA,
            pltpu.SemaphoreType.DMA,
            pltpu.SemaphoreType.DMA,
            pltpu.SemaphoreType.DMA,
            pltpu.SemaphoreType.DMA,
            pltpu.SemaphoreType.DMA,
            pltpu.SemaphoreType.DMA,
        ],
    )
    def k(h_hbm, er_hbm, src_hbm, dst_hbm, out_hbm,
          sib0, sib1, dib0, dib1, gb0, gb1, eb0, eb1, zbuf, acc,
          ssi0, ssi1, sdi0, sdi1, sg0, sg1, sel0, sel1, ss):
        cid = lax.axis_index("c")
        sid = lax.axis_index("s")
        sibs, dibs = (sib0, sib1), (dib0, dib1)
        gbs, ebs = (gb0, gb1), (eb0, eb1)
        ssis, sdis = (ssi0, ssi1), (sdi0, sdi1)
        sgs, sels = (sg0, sg1), (sel0, sel1)

        @pl.loop(0, zrows)
        def _(r):
            for cc in range(8):
                zbuf[r, pl.ds(cc * 16, 16)] = jnp.zeros((16,), F32)

        for kk in range(rows_per_sub // zrows):
            pltpu.sync_copy(
                zbuf, acc.at[pl.ds(sid * rows_per_sub + kk * zrows, zrows)])
        plsc.subcore_barrier()

        base = cid * e2 + sid * per_w
        pltpu.sync_copy(src_hbm.at[pl.ds(base, chunk)], sib0)
        pltpu.sync_copy(dst_hbm.at[pl.ds(base, chunk)], dib0)
        pltpu.async_copy(h_hbm.at[sib0], gb0, sg0)
        pltpu.async_copy(er_hbm.at[pl.ds(base, chunk)], eb0, sel0)

        @pl.loop(0, n_chunks)
        def _(ci):
            off = base + ci * chunk
            for s in range(2):
                s2 = 1 - s

                @pl.when(ci % 2 == s)
                def _(s=s, s2=s2):
                    @pl.when(ci >= 1)
                    def _():
                        pltpu.make_async_copy(
                            h_hbm.at[pl.ds(0, chunk)], gbs[s2], ss).wait()

                    @pl.when(ci + 1 < n_chunks)
                    def _():
                        pltpu.async_copy(
                            src_hbm.at[pl.ds(off + chunk, chunk)],
                            sibs[s2], ssis[s2])
                        pltpu.async_copy(
                            dst_hbm.at[pl.ds(off + chunk, chunk)],
                            dibs[s2], sdis[s2])
                        pltpu.async_copy(
                            er_hbm.at[pl.ds(off + chunk, chunk)],
                            ebs[s2], sels[s2])

                    pltpu.make_async_copy(
                        h_hbm.at[pl.ds(0, chunk)], gbs[s], sgs[s]).wait()
                    pltpu.make_async_copy(
                        er_hbm.at[pl.ds(0, chunk)], ebs[s], sels[s]).wait()

                    @plsc.parallel_loop(0, chunk, unroll=4)
                    def _(ei, s=s):
                        for cc in range(8):
                            g = gbs[s][ei, pl.ds(cc * 16, 16)]
                            ee = ebs[s][ei, pl.ds(cc * 16, 16)]
                            gbs[s][ei, pl.ds(cc * 16, 16)] = (
                                jnp.maximum(g + ee, 0.0))

                    pltpu.async_copy(gbs[s], acc.at[dibs[s]], ss, add=True)

                    @pl.when(ci + 1 < n_chunks)
                    def _():
                        pltpu.make_async_copy(
                            src_hbm.at[pl.ds(0, chunk)],
                            sibs[s2], ssis[s2]).wait()
                        pltpu.make_async_copy(
                            dst_hbm.at[pl.ds(0, chunk)],
                            dibs[s2], sdis[s2]).wait()
                        pltpu.async_copy(h_hbm.at[sibs[s2]], gbs[s2], sgs[s2])

        ls = (n_chunks - 1) % 2
        pltpu.make_async_copy(
            h_hbm.at[pl.ds(0, chunk)], gbs[ls], ss).wait()

        plsc.subcore_barrier()
        for kk in range(rows_per_sub // zrows):
            rs = sid * rows_per_sub + kk * zrows
            pltpu.sync_copy(acc.at[pl.ds(rs, zrows)],
                            out_hbm.at[cid].at[pl.ds(rs, zrows)])

    return k(h, er, src, dst)[:, :n, :]


def _sc_scatter_add(values, idx, n, chunk=80):
    e, width = values.shape
    e2 = e // NC
    per_w = e2 // NS
    n_chunks = per_w // chunk
    npad = 10240
    rows_per_sub = npad // NS
    zrows = rows_per_sub // 5

    @functools.partial(
        pl.kernel,
        out_type=jax.ShapeDtypeStruct((NC, npad, width), F32),
        mesh=_MESH,
        scratch_types=[
            pltpu.VMEM((per_w,), jnp.int32),
            pltpu.VMEM((chunk,), jnp.int32),
            pltpu.VMEM((chunk,), jnp.int32),
            pltpu.VMEM((chunk, width), F32),
            pltpu.VMEM((chunk, width), F32),
            pltpu.VMEM((zrows, width), F32),
            pltpu.VMEM_SHARED((npad, width), F32),
            pltpu.SemaphoreType.DMA,
            pltpu.SemaphoreType.DMA,
            pltpu.SemaphoreType.DMA,
            pltpu.SemaphoreType.DMA,
        ],
    )
    def k(val_hbm, idx_hbm, out_hbm, idx_all, ib0, ib1, vb0, vb1, zbuf,
          acc, sl0, sl1, ss0, ss1):
        cid = lax.axis_index("c")
        sid = lax.axis_index("s")
        ibs, vbs, sls, sss = (ib0, ib1), (vb0, vb1), (sl0, sl1), (ss0, ss1)

        @pl.loop(0, zrows)
        def _(r):
            for cc in range(width // 16):
                zbuf[r, pl.ds(cc * 16, 16)] = jnp.zeros((16,), F32)

        for kk in range(5):
            pltpu.sync_copy(
                zbuf, acc.at[pl.ds(sid * rows_per_sub + kk * zrows, zrows)])
        plsc.subcore_barrier()

        base = cid * e2 + sid * per_w
        pltpu.sync_copy(idx_hbm.at[pl.ds(base, per_w)], idx_all)
        pltpu.async_copy(val_hbm.at[pl.ds(base, chunk)], vb0, sl0)

        @pl.loop(0, n_chunks)
        def _(ci):
            off = base + ci * chunk
            for s in range(2):
                s2 = 1 - s

                @pl.when(ci % 2 == s)
                def _(s=s, s2=s2):
                    @pl.when(ci >= 1)
                    def _():
                        pltpu.make_async_copy(
                            val_hbm.at[pl.ds(off, chunk)], vbs[s2], sss[s2]
                        ).wait()

                    @pl.when(ci + 1 < n_chunks)
                    def _():
                        pltpu.async_copy(
                            val_hbm.at[pl.ds(off + chunk, chunk)],
                            vbs[s2], sls[s2])

                    pltpu.make_async_copy(
                        val_hbm.at[pl.ds(off, chunk)], vbs[s], sls[s]
                    ).wait()
                    for o in range(0, chunk, 16):
                        ibs[s][pl.ds(o, 16)] = (
                            idx_all[pl.ds(ci * chunk + o, 16)])
                    pltpu.async_copy(
                        vbs[s], acc.at[ibs[s]], sss[s], add=True)

        ls = (n_chunks - 1) % 2
        pltpu.make_async_copy(
            val_hbm.at[pl.ds(base, chunk)], vbs[ls], sss[ls]).wait()

        plsc.subcore_barrier()
        for kk in range(5):
            rs = sid * rows_per_sub + kk * zrows
            pltpu.sync_copy(acc.at[pl.ds(rs, zrows)],
                            out_hbm.at[cid].at[pl.ds(rs, zrows)])

    return k(values, idx)[:, :n, :]


def _sc_segmax(score_flat, idx, n, chunk=1000):
    e = idx.shape[0]
    per_w = e // NW
    n_chunks = per_w // chunk
    tsz = n * 8 + 16

    @functools.partial(
        pl.kernel,
        out_type=jax.ShapeDtypeStruct((NW * n * 8,), F32),
        mesh=_MESH,
        scratch_types=[
            pltpu.VMEM((chunk + 16,), jnp.int32),
            pltpu.VMEM((chunk * 16,), F32),
            pltpu.VMEM((tsz,), F32),
        ],
        compiler_params=pltpu.CompilerParams(needs_layout_passes=False),
    )
    def k(score_hbm, idx_hbm, out_hbm, idxbuf, scorebuf, table):
        cid = lax.axis_index("c")
        sid = lax.axis_index("s")
        wid = cid * NS + sid
        base = wid * per_w

        @pl.loop(0, tsz, step=16)
        def _(i):
            table[pl.ds(i, 16)] = jnp.full((16,), -3.0e38, F32)

        lanes = lax.iota(jnp.int32, 16)
        mask = lanes < 8

        @pl.loop(0, n_chunks)
        def _(ci):
            off = base + ci * chunk
            pltpu.sync_copy(score_hbm.at[pl.ds(off * 16, chunk * 16)],
                            scorebuf)
            pltpu.sync_copy(idx_hbm.at[pl.ds(off, chunk)],
                            idxbuf.at[pl.ds(0, chunk)])

            @pl.loop(0, chunk)
            def _(ei):
                d = idxbuf[pl.ds(ei, 16)][0]
                sv = scorebuf[pl.ds(ei * 16, 16)]
                iv = d * 8 + lanes
                cur = plsc.load_gather(table, [iv], mask=mask)
                plsc.store_scatter(table, [iv],
                                   jnp.maximum(cur, sv), mask=mask)

        pltpu.sync_copy(table.at[pl.ds(0, n * 8)],
                        out_hbm.at[pl.ds(wid * n * 8, n * 8)])

    return k(score_flat, idx)


def _sc_den_exp(score_flat, smax_pad, idx, n, chunk=80):
    e = idx.shape[0]
    e2 = e // NC
    per_w = e2 // NS
    n_chunks = per_w // chunk
    npad = 10240
    rows_per_sub = npad // NS
    zrows = 32

    @functools.partial(
        pl.kernel,
        out_type=[
            jax.ShapeDtypeStruct((NC, npad, 128), F32),
            jax.ShapeDtypeStruct((e * 16,), F32),
        ],
        mesh=_MESH,
        scratch_types=[
            pltpu.VMEM((per_w,), jnp.int32),
            pltpu.VMEM((chunk,), jnp.int32),
            pltpu.VMEM((chunk,), jnp.int32),
            pltpu.VMEM((chunk * 16,), F32),
            pltpu.VMEM((chunk * 16,), F32),
            pltpu.VMEM((chunk, 128), F32),
            pltpu.VMEM((chunk, 128), F32),
            pltpu.VMEM((chunk, 128), F32),
            pltpu.VMEM((chunk * 16,), F32),
            pltpu.VMEM((zrows, 128), F32),
            pltpu.VMEM_SHARED((npad, 128), F32),
            pltpu.SemaphoreType.DMA,
            pltpu.SemaphoreType.DMA,
            pltpu.SemaphoreType.DMA,
            pltpu.SemaphoreType.DMA,
            pltpu.SemaphoreType.DMA,
            pltpu.SemaphoreType.DMA,
        ],
    )
    def k(score_hbm, smax_hbm, idx_hbm, out_hbm, ex_hbm,
          idx_all, ib0, ib1, sb0, sb1, sm0, sm1, vb, eb,
          zbuf, acc, sl0, sl1, sg0, sg1, ss, sw):
        cid = lax.axis_index("c")
        sid = lax.axis_index("s")
        ibs, sbs, sms = (ib0, ib1), (sb0, sb1), (sm0, sm1)
        sls, sgs = (sl0, sl1), (sg0, sg1)

        @pl.loop(0, zrows)
        def _(r):
            for cc in range(8):
                zbuf[r, pl.ds(cc * 16, 16)] = jnp.zeros((16,), F32)

        for kk in range(rows_per_sub // zrows):
            pltpu.sync_copy(
                zbuf, acc.at[pl.ds(sid * rows_per_sub + kk * zrows, zrows)])
        plsc.subcore_barrier()

        @pl.loop(0, chunk)
        def _(r):
            for cc in range(1, 8):
                vb[r, pl.ds(cc * 16, 16)] = jnp.zeros((16,), F32)

        base = cid * e2 + sid * per_w
        pltpu.sync_copy(idx_hbm.at[pl.ds(base, per_w)], idx_all)
        for o in range(0, chunk, 16):
            ib0[pl.ds(o, 16)] = idx_all[pl.ds(o, 16)]
        pltpu.async_copy(
            score_hbm.at[pl.ds(base * 16, chunk * 16)], sb0, sl0)
        pltpu.async_copy(smax_hbm.at[ib0], sm0, sg0)

        @pl.loop(0, n_chunks)
        def _(ci):
            off = base + ci * chunk
            for s in range(2):
                s2 = 1 - s

                @pl.when(ci % 2 == s)
                def _(s=s, s2=s2):
                    pltpu.make_async_copy(
                        score_hbm.at[pl.ds(off * 16, chunk * 16)],
                        sbs[s], sls[s]).wait()
                    pltpu.make_async_copy(
                        smax_hbm.at[pl.ds(0, chunk)], sms[s], sgs[s]).wait()

                    @pl.when(ci >= 1)
                    def _():
                        pltpu.make_async_copy(
                            smax_hbm.at[pl.ds(0, chunk)], vb, ss).wait()
                        pltpu.make_async_copy(
                            score_hbm.at[pl.ds(off * 16, chunk * 16)],
                            eb, sw).wait()

                    @pl.when(ci + 1 < n_chunks)
                    def _():
                        for o in range(0, chunk, 16):
                            ibs[s2][pl.ds(o, 16)] = (
                                idx_all[pl.ds((ci + 1) * chunk + o, 16)])
                        pltpu.async_copy(
                            score_hbm.at[
                                pl.ds((off + chunk) * 16, chunk * 16)],
                            sbs[s2], sls[s2])
                        pltpu.async_copy(
                            smax_hbm.at[ibs[s2]], sms[s2], sgs[s2])

                    @plsc.parallel_loop(0, chunk, unroll=4)
                    def _(ei, s=s):
                        sv = sbs[s][pl.ds(ei * 16, 16)]
                        m = sms[s][ei, pl.ds(0, 16)]
                        ex = jnp.exp(sv - m)
                        vb[ei, pl.ds(0, 16)] = ex
                        eb[pl.ds(ei * 16, 16)] = ex

                    pltpu.async_copy(vb, acc.at[ibs[s]], ss, add=True)
                    pltpu.async_copy(
                        eb, ex_hbm.at[pl.ds(off * 16, chunk * 16)], sw)

        pltpu.make_async_copy(
            smax_hbm.at[pl.ds(0, chunk)], vb, ss).wait()
        pltpu.make_async_copy(
            score_hbm.at[pl.ds(base * 16, chunk * 16)], eb, sw).wait()

        plsc.subcore_barrier()
        for kk in range(rows_per_sub // zrows):
            rs = sid * rows_per_sub + kk * zrows
            pltpu.sync_copy(acc.at[pl.ds(rs, zrows)],
                            out_hbm.at[cid].at[pl.ds(rs, zrows)])

    den, ex_flat = k(score_flat, smax_pad, idx)
    return den[:, :n, :], ex_flat


def _tc_wv(ex_pk, vs3, be=3200):
    ep8 = ex_pk.shape[0]
    bp = be // 8

    def body(p_r, vs_r, o_r):
        p = p_r[...]
        a = lax.broadcasted_iota(jnp.int32, (128, 128), 0)
        j = lax.broadcasted_iota(jnp.int32, (128, 128), 1)
        parts = []
        for g in range(8):
            m = (a == g * 16 + j // 16).astype(F32)
            parts.append(_dot(p, m)[:, None, :])
        exb = jnp.concatenate(parts, axis=1)
        o_r[...] = vs_r[...] * exb

    return pl.pallas_call(
        body,
        grid=(ep8 // bp,),
        in_specs=[
            pl.BlockSpec((bp, 128), lambda i: (i, 0)),
            pl.BlockSpec((bp, 8, 128), lambda i: (i, 0, 0)),
        ],
        out_specs=pl.BlockSpec((bp, 8, 128), lambda i: (i, 0, 0)),
        out_shape=jax.ShapeDtypeStruct((ep8, 8, 128), F32),
    )(ex_pk, vs3)


def kernel(x, edge_index, dist_rbf, eig_pe, edge_attr, batch,
           W_phi1, b_phi1, W_phi2, b_phi2, W_edge, W_rho, b_rho,
           Wq, Wk, Wv, We, Wr1, br1, Wr2, br2):
    n = x.shape[0]
    e = edge_index.shape[1]
    src = edge_index[0]
    dst = edge_index[1]
    b_phi1 = b_phi1.reshape(1, -1)
    b_phi2 = b_phi2.reshape(1, -1)
    b_rho = b_rho.reshape(1, -1)
    br1 = br1.reshape(1, -1)
    br2 = br2.reshape(1, 1)
    batch3 = batch.reshape(n // 1000, 1, 1000)

    h = _tc_signnet_h(eig_pe, W_phi1, b_phi1, W_phi2, b_phi2)
    er = _tc_er(edge_attr, W_edge)
    m_part = _sc_msg_fused(h, er, src, dst, n)
    q, k, v = _tc_qkv(x, h, m_part, W_rho, b_rho, Wq, Wk, Wv)
    qd, ks, vs = _sc_gather([q, k, v], [dst, src, src], chunk=80)
    score = _tc_score(qd, ks, dist_rbf, We)
    score_flat = score.reshape(-1)
    smax_part = _sc_segmax(score_flat, dst, n)
    smax_pad = _tc_smax_merge(
        smax_part.reshape(NW, n // 16, 128)).reshape(n, 128)
    den_part, ex_flat = _sc_den_exp(score_flat, smax_pad, dst, n)
    wv = _tc_wv(ex_flat.reshape(e // 8, 128),
                vs.reshape(e // 8, 8, 128)).reshape(e, 128)
    agg_part = _sc_scatter_add(wv, dst, n)
    out = _tc_final(agg_part, den_part, batch3, Wr1, br1, Wr2, br2)
    return out
